# Initial kernel scaffold; baseline (speedup 1.0000x reference)
import jax
import jax.numpy as jnp
from jax.experimental import pallas as pl


def kernel(features_1, edge_index_1, features_2, edge_index_2, W1, b1, W2, b2, W3, b3, Wm1, Wm2, Wa, Wt, Wb, bt, Wfc, bfc, Ws, bs):
    raise NotImplementedError("write your pallas kernel here")


# speedup vs baseline: 10.1641x; 10.1641x over previous
import functools

import jax
import jax.numpy as jnp
from jax import lax
from jax.experimental import pallas as pl
from jax.experimental.pallas import tpu as pltpu
from jax.experimental.pallas import tpu_sc as plsc

NN = 50000
EE = 800000
NPAD = 51200
EPAD = 819200
ECH = EPAD // 128
RPT = NPAD // 16
SCH = 8
BLK = 400
NBLK = NN // BLK
F32 = jnp.float32

@functools.cache
def _mesh():
    return plsc.VectorSubcoreMesh(core_axis_name="c", subcore_axis_name="s",
                                  num_cores=2, num_subcores=16)


def _zero_fill(buf, width):
    z16 = jnp.zeros((16,), F32)

    def body(i, _):
        for h in range(width // 16):
            buf[i, pl.ds(16 * h, 16)] = z16
        return 0

    lax.fori_loop(0, buf.shape[0], body, 0)


def _unit_scatter(src2d, dst2d, table, out, acc, sbuf, dbuf, rows,
                  gsem, tile, chunk0, ntile_chunks):
    r0 = tile * RPT

    _zero_fill(rows[0], rows[0].shape[1])

    def zb(i, _):
        pltpu.sync_copy(rows[0], acc.at[pl.ds(r0 + i * 128, 128)])
        return 0

    lax.fori_loop(0, RPT // 128, zb, 0)
    plsc.subcore_barrier()

    nsuper = ntile_chunks // SCH

    def sbody(si, _):
        cb = chunk0 + si * SCH
        pltpu.sync_copy(src2d.at[pl.ds(cb, SCH)], sbuf)
        pltpu.sync_copy(dst2d.at[pl.ds(cb, SCH)], dbuf)
        pend = pltpu.async_copy(table.at[sbuf.at[0]], rows[0], gsem)
        for j in range(SCH):
            pend.wait()
            if j + 1 < SCH:
                pend = pltpu.async_copy(table.at[sbuf.at[j + 1]],
                                        rows[(j + 1) % 2], gsem)
            pltpu.sync_copy(rows[j % 2], acc.at[dbuf.at[j]], add=True)
        return 0

    lax.fori_loop(0, nsuper, sbody, 0)
    plsc.subcore_barrier()

    def wb(i, _):
        pltpu.sync_copy(acc.at[pl.ds(r0 + i * 128, 128)], rows[0])
        pltpu.sync_copy(rows[0], out.at[pl.ds(r0 + i * 128, 128)])
        return 0

    lax.fori_loop(0, RPT // 128, wb, 0)


def _unit_deg(dst2d, out, acc, dbuf, obuf, tile, chunk0, ntile_chunks):
    r0 = tile * RPT

    def zb(i, _):
        pltpu.sync_copy(obuf, acc.at[pl.ds(r0 + i * 128, 128)])
        return 0

    lax.fori_loop(0, RPT // 128, zb, 0)
    plsc.subcore_barrier()

    one16 = jnp.ones((16,), F32)

    def ob(i, _):
        obuf[i, pl.ds(0, 16)] = one16
        return 0

    lax.fori_loop(0, 128, ob, 0)

    nsuper = ntile_chunks // SCH

    def sbody(si, _):
        cb = chunk0 + si * SCH
        pltpu.sync_copy(dst2d.at[pl.ds(cb, SCH)], dbuf)
        for j in range(SCH):
            pltpu.sync_copy(obuf, acc.at[dbuf.at[j]], add=True)
        return 0

    lax.fori_loop(0, nsuper, sbody, 0)
    plsc.subcore_barrier()

    def wb(i, _):
        pltpu.sync_copy(acc.at[pl.ds(r0 + i * 128, 128)], obuf)
        pltpu.sync_copy(obuf, out.at[pl.ds(r0 + i * 128, 128)])
        return 0

    lax.fori_loop(0, RPT // 128, wb, 0)


@functools.cache
def _make_deg():
    @functools.partial(
        pl.kernel,
        out_type=[jax.ShapeDtypeStruct((NPAD, 16), F32)] * 2,
        mesh=_mesh(),
        compiler_params=pltpu.CompilerParams(use_tc_tiling_on_sc=False),
        scratch_types=[
            pltpu.VMEM_SHARED((NPAD + 16, 16), F32),
            pltpu.VMEM((SCH, 128), jnp.int32),
            pltpu.VMEM((128, 16), F32),
        ],
    )
    def _deg_kernel(dst1, dst2, deg1, deg2, acc, dbuf, obuf):
        c = lax.axis_index("c")
        s = lax.axis_index("s")
        _zero_fill(obuf, 16)

        @pl.when(c == 0)
        def _():
            _unit_deg(dst1, deg1, acc, dbuf, obuf, s, s * 400, 400)

        @pl.when(c == 1)
        def _():
            _unit_deg(dst2, deg2, acc, dbuf, obuf, s, s * 400, 400)

    return _deg_kernel


@functools.cache
def _make_layer12():
    @functools.partial(
        pl.kernel,
        out_type=[jax.ShapeDtypeStruct((NPAD, 32), F32)] * 4,
        mesh=_mesh(),
        compiler_params=pltpu.CompilerParams(use_tc_tiling_on_sc=False),
        scratch_types=[
            pltpu.VMEM_SHARED((NPAD + 16, 32), F32),
            pltpu.VMEM((SCH, 128), jnp.int32),
            pltpu.VMEM((SCH, 128), jnp.int32),
            pltpu.VMEM((128, 32), F32),
            pltpu.VMEM((128, 32), F32),
            pltpu.SemaphoreType.DMA,
        ],
    )
    def _k(src1, dst1, src2, dst2, g1h0, g1h1, g2h0, g2h1,
           o1h0, o1h1, o2h0, o2h1,
           acc, sbuf, dbuf, rows0, rows1, gsem):
        c = lax.axis_index("c")
        s = lax.axis_index("s")
        rows = (rows0, rows1)

        @pl.when(c == 0)
        def _():
            _unit_scatter(src1, dst1, g1h0, o1h0, acc, sbuf, dbuf, rows,
                          gsem, s, s * 400, 400)
            _unit_scatter(src2, dst2, g2h0, o2h0, acc, sbuf, dbuf, rows,
                          gsem, s, s * 400, 400)

        @pl.when(c == 1)
        def _():
            _unit_scatter(src1, dst1, g1h1, o1h1, acc, sbuf, dbuf, rows,
                          gsem, s, s * 400, 400)
            _unit_scatter(src2, dst2, g2h1, o2h1, acc, sbuf, dbuf, rows,
                          gsem, s, s * 400, 400)

    return _k


@functools.cache
def _make_layer3():
    @functools.partial(
        pl.kernel,
        out_type=[jax.ShapeDtypeStruct((NPAD, 32), F32)] * 4,
        mesh=_mesh(),
        compiler_params=pltpu.CompilerParams(use_tc_tiling_on_sc=False),
        scratch_types=[
            pltpu.VMEM_SHARED((NPAD + 16, 32), F32),
            pltpu.VMEM((SCH, 128), jnp.int32),
            pltpu.VMEM((SCH, 128), jnp.int32),
            pltpu.VMEM((128, 32), F32),
            pltpu.VMEM((128, 32), F32),
            pltpu.SemaphoreType.DMA,
        ],
    )
    def _layer3_kernel(src1, dst1, src2, dst2, g3a, g3b,
                       pa0, pa1, pb0, pb1,
                       acc, sbuf, dbuf, rows0, rows1, gsem):
        c = lax.axis_index("c")
        s = lax.axis_index("s")
        rows = (rows0, rows1)

        @pl.when(c == 0)
        def _():
            _unit_scatter(src1, dst1, g3a, pa0, acc, sbuf, dbuf, rows,
                          gsem, s, s * 200, 200)
            _unit_scatter(src2, dst2, g3b, pb0, acc, sbuf, dbuf, rows,
                          gsem, s, s * 200, 200)

        @pl.when(c == 1)
        def _():
            _unit_scatter(src1, dst1, g3a, pa1, acc, sbuf, dbuf, rows,
                          gsem, s, 3200 + s * 200, 200)
            _unit_scatter(src2, dst2, g3b, pb1, acc, sbuf, dbuf, rows,
                          gsem, s, 3200 + s * 200, 200)

    return _layer3_kernel


def _mm1_body(x_ref, w_ref, deg_ref, o0_ref, o1_ref):
    dinv = lax.rsqrt(deg_ref[:, 0:1] + 1.0)
    h = jnp.dot(x_ref[...], w_ref[...], preferred_element_type=F32) * dinv
    o0_ref[...] = h[:, :32]
    o1_ref[...] = h[:, 32:]


def _mm1(x, W, deg):
    return pl.pallas_call(
        _mm1_body,
        grid=(NBLK,),
        in_specs=[
            pl.BlockSpec((BLK, 128), lambda i: (i, 0)),
            pl.BlockSpec((128, 64), lambda i: (0, 0)),
            pl.BlockSpec((BLK, 16), lambda i: (i, 0)),
        ],
        out_specs=[pl.BlockSpec((BLK, 32), lambda i: (i, 0))] * 2,
        out_shape=[jax.ShapeDtypeStruct((NN, 32), F32)] * 2,
    )(x, W, deg)


def _mm_mid_body2(s0, s1, g0, g1, deg, b, w, o0, o1):
    dinv = lax.rsqrt(deg[:, 0:1] + 1.0)
    prev = jnp.concatenate([s0[...] + g0[...], s1[...] + g1[...]], axis=1)
    x = jnp.maximum(prev * dinv + b[...], 0.0)
    h = jnp.dot(x, w[...], preferred_element_type=F32) * dinv
    o0[...] = h[:, :32]
    o1[...] = h[:, 32:]


def _mm_mid_body1(s0, s1, g0, g1, deg, b, w, o0):
    dinv = lax.rsqrt(deg[:, 0:1] + 1.0)
    prev = jnp.concatenate([s0[...] + g0[...], s1[...] + g1[...]], axis=1)
    x = jnp.maximum(prev * dinv + b[...], 0.0)
    o0[...] = jnp.dot(x, w[...], preferred_element_type=F32) * dinv


def _mm_mid(s0, s1, g0, g1, deg, b_row, W, halves):
    fout = W.shape[1]
    nouts = 2 if halves else 1
    return pl.pallas_call(
        _mm_mid_body2 if halves else _mm_mid_body1,
        grid=(NBLK,),
        in_specs=[
            pl.BlockSpec((BLK, 32), lambda i: (i, 0)),
            pl.BlockSpec((BLK, 32), lambda i: (i, 0)),
            pl.BlockSpec((BLK, 32), lambda i: (i, 0)),
            pl.BlockSpec((BLK, 32), lambda i: (i, 0)),
            pl.BlockSpec((BLK, 16), lambda i: (i, 0)),
            pl.BlockSpec((1, 64), lambda i: (0, 0)),
            pl.BlockSpec((64, fout), lambda i: (0, 0)),
        ],
        out_specs=[pl.BlockSpec((BLK, fout // nouts), lambda i: (i, 0))] * nouts,
        out_shape=[jax.ShapeDtypeStruct((NN, fout // nouts), F32)] * nouts,
    )(s0, s1, g0, g1, deg, b_row, W)


def _t1_body(p10, p11, g31, deg1, p20, p21, g32, deg2, b3, wm1, wm2,
             a1_ref, a2_ref, m1_ref, m2_ref, acc1, acc2):
    i = pl.program_id(0)

    @pl.when(i == 0)
    def _():
        acc1[...] = jnp.zeros_like(acc1)
        acc2[...] = jnp.zeros_like(acc2)

    dinv1 = lax.rsqrt(deg1[:, 0:1] + 1.0)
    a1 = (p10[...] + p11[...] + g31[...]) * dinv1 + b3[...]
    dinv2 = lax.rsqrt(deg2[:, 0:1] + 1.0)
    a2 = (p20[...] + p21[...] + g32[...]) * dinv2 + b3[...]
    a1_ref[...] = a1
    a2_ref[...] = a2
    acc1[...] += jnp.sum(a1, axis=0, keepdims=True)
    acc2[...] += jnp.sum(a2, axis=0, keepdims=True)
    inv_n = 1.0 / NN
    m1_ref[...] = jnp.tanh(
        jnp.dot(acc1[...] * inv_n, wm1[...], preferred_element_type=F32))
    m2_ref[...] = jnp.tanh(
        jnp.dot(acc2[...] * inv_n, wm2[...], preferred_element_type=F32))


def _t1(p10, p11, g31, deg1, p20, p21, g32, deg2, b3_row, Wm1, Wm2):
    blk = pl.BlockSpec((BLK, 32), lambda i: (i, 0))
    small = pl.BlockSpec((1, 32), lambda i: (0, 0))
    return pl.pallas_call(
        _t1_body,
        grid=(NBLK,),
        in_specs=[blk, blk, blk, pl.BlockSpec((BLK, 16), lambda i: (i, 0)),
                  blk, blk, blk, pl.BlockSpec((BLK, 16), lambda i: (i, 0)),
                  small,
                  pl.BlockSpec((32, 32), lambda i: (0, 0)),
                  pl.BlockSpec((32, 32), lambda i: (0, 0))],
        out_specs=[blk, blk, small, small],
        out_shape=[jax.ShapeDtypeStruct((NN, 32), F32),
                   jax.ShapeDtypeStruct((NN, 32), F32),
                   jax.ShapeDtypeStruct((1, 32), F32),
                   jax.ShapeDtypeStruct((1, 32), F32)],
        scratch_shapes=[pltpu.VMEM((1, 32), F32), pltpu.VMEM((1, 32), F32)],
    )(p10, p11, g31, deg1, p20, p21, g32, deg2, b3_row, Wm1, Wm2)


def _t2_body(a1, a2, m1, m2, wa, ctx1_ref, ctx2_ref, acc1, acc2):
    i = pl.program_id(0)

    @pl.when(i == 0)
    def _():
        acc1[...] = jnp.zeros_like(acc1)
        acc2[...] = jnp.zeros_like(acc2)

    x1 = jnp.abs(a1[...] - m2[...])
    x2 = jnp.abs(a2[...] - m1[...])
    acc1[...] += jnp.sum(x1, axis=0, keepdims=True)
    acc2[...] += jnp.sum(x2, axis=0, keepdims=True)
    inv_n = 1.0 / NN
    ctx1_ref[...] = jnp.tanh(
        jnp.dot(acc1[...] * inv_n, wa[...], preferred_element_type=F32))
    ctx2_ref[...] = jnp.tanh(
        jnp.dot(acc2[...] * inv_n, wa[...], preferred_element_type=F32))


def _t2(a1, a2, m1, m2, Wa):
    blk = pl.BlockSpec((BLK, 32), lambda i: (i, 0))
    small = pl.BlockSpec((1, 32), lambda i: (0, 0))
    return pl.pallas_call(
        _t2_body,
        grid=(NBLK,),
        in_specs=[blk, blk, small, small,
                  pl.BlockSpec((32, 32), lambda i: (0, 0))],
        out_specs=[small, small],
        out_shape=[jax.ShapeDtypeStruct((1, 32), F32)] * 2,
        scratch_shapes=[pltpu.VMEM((1, 32), F32), pltpu.VMEM((1, 32), F32)],
    )(a1, a2, m1, m2, Wa)


def _t3_body(a1, a2, m1, m2, ctx1, ctx2, p1_ref, p2_ref, acc1, acc2):
    i = pl.program_id(0)

    @pl.when(i == 0)
    def _():
        acc1[...] = jnp.zeros_like(acc1)
        acc2[...] = jnp.zeros_like(acc2)

    x1 = jnp.abs(a1[...] - m2[...])
    x2 = jnp.abs(a2[...] - m1[...])
    s1 = jax.nn.sigmoid(jnp.sum(x1 * ctx1[...], axis=1, keepdims=True))
    s2 = jax.nn.sigmoid(jnp.sum(x2 * ctx2[...], axis=1, keepdims=True))
    acc1[...] += jnp.sum(x1 * s1, axis=0, keepdims=True)
    acc2[...] += jnp.sum(x2 * s2, axis=0, keepdims=True)
    p1_ref[...] = acc1[...]
    p2_ref[...] = acc2[...]


def _t3(a1, a2, m1, m2, ctx1, ctx2):
    blk = pl.BlockSpec((BLK, 32), lambda i: (i, 0))
    small = pl.BlockSpec((1, 32), lambda i: (0, 0))
    return pl.pallas_call(
        _t3_body,
        grid=(NBLK,),
        in_specs=[blk, blk, small, small, small, small],
        out_specs=[small, small],
        out_shape=[jax.ShapeDtypeStruct((1, 32), F32)] * 2,
        scratch_shapes=[pltpu.VMEM((1, 32), F32), pltpu.VMEM((1, 32), F32)],
    )(a1, a2, m1, m2, ctx1, ctx2)


def _t4_body(p1, p2, wtt, wbt, btr, wfc, bfcr, ws, bsr, out_ref, sc):
    t = pl.program_id(0)
    a = wtt[0]
    v = jnp.dot(p1[...], a, preferred_element_type=F32)
    s_t = jnp.sum(v * p2[...])
    iota = lax.broadcasted_iota(jnp.int32, (1, 16), 1)
    base = jnp.where(t == 0, jnp.zeros_like(sc[...]), sc[...])
    sc[...] = jnp.where(iota == t, s_t, base)
    combined = jnp.concatenate([p1[...], p2[...]], axis=1)
    block = jnp.dot(combined, wbt[...], preferred_element_type=F32)
    scores = jnp.maximum(sc[...] + block + btr[...], 0.0)
    h = jnp.maximum(
        jnp.dot(scores, wfc[...], preferred_element_type=F32) + bfcr[...], 0.0)
    out_ref[...] = jax.nn.sigmoid(
        jnp.dot(h, ws[...], preferred_element_type=F32) + bsr[...])


def _t4(p1, p2, Wtt, Wbt, btr, Wfc, bfcr, Ws, bsr):
    small = pl.BlockSpec((1, 32), lambda t: (0, 0))
    return pl.pallas_call(
        _t4_body,
        grid=(16,),
        in_specs=[small, small,
                  pl.BlockSpec((1, 32, 32), lambda t: (t, 0, 0)),
                  pl.BlockSpec((64, 16), lambda t: (0, 0)),
                  pl.BlockSpec((1, 16), lambda t: (0, 0)),
                  pl.BlockSpec((16, 16), lambda t: (0, 0)),
                  pl.BlockSpec((1, 16), lambda t: (0, 0)),
                  pl.BlockSpec((16, 1), lambda t: (0, 0)),
                  pl.BlockSpec((1, 1), lambda t: (0, 0))],
        out_specs=pl.BlockSpec((1, 1), lambda t: (0, 0)),
        out_shape=jax.ShapeDtypeStruct((1, 1), F32),
        scratch_shapes=[pltpu.VMEM((1, 16), F32)],
    )(p1, p2, Wtt, Wbt, btr, Wfc, bfcr, Ws, bsr)


def _pad_edges(ei):
    pad = EPAD - EE
    src = jnp.concatenate([ei[0], jnp.zeros((pad,), jnp.int32)])
    dst = jnp.concatenate([ei[1], jnp.full((pad,), NPAD, jnp.int32)])
    return src.reshape(ECH, 128), dst.reshape(ECH, 128)


def kernel(features_1, edge_index_1, features_2, edge_index_2, W1, b1, W2, b2,
           W3, b3, Wm1, Wm2, Wa, Wt, Wb, bt, Wfc, bfc, Ws, bs):
    src1, dst1 = _pad_edges(edge_index_1)
    src2, dst2 = _pad_edges(edge_index_2)

    deg1, deg2 = _make_deg()(dst1, dst2)

    b1r = b1.reshape(1, 64)
    b2r = b2.reshape(1, 64)
    b3r = b3.reshape(1, 32)

    g1h0, g1h1 = _mm1(features_1, W1, deg1)
    g2h0, g2h1 = _mm1(features_2, W1, deg2)
    s1h0, s1h1, s2h0, s2h1 = _make_layer12()(src1, dst1, src2, dst2,
                                             g1h0, g1h1, g2h0, g2h1)

    q1h0, q1h1 = _mm_mid(s1h0, s1h1, g1h0, g1h1, deg1, b1r, W2, True)
    q2h0, q2h1 = _mm_mid(s2h0, s2h1, g2h0, g2h1, deg2, b1r, W2, True)
    t1h0, t1h1, t2h0, t2h1 = _make_layer12()(src1, dst1, src2, dst2,
                                             q1h0, q1h1, q2h0, q2h1)

    (g31,) = _mm_mid(t1h0, t1h1, q1h0, q1h1, deg1, b2r, W3, False)
    (g32,) = _mm_mid(t2h0, t2h1, q2h0, q2h1, deg2, b2r, W3, False)
    pa0, pa1, pb0, pb1 = _make_layer3()(src1, dst1, src2, dst2, g31, g32)

    a1, a2, m1, m2 = _t1(pa0, pa1, g31, deg1, pb0, pb1, g32, deg2,
                         b3r, Wm1, Wm2)
    ctx1, ctx2 = _t2(a1, a2, m1, m2, Wa)
    p1, p2 = _t3(a1, a2, m1, m2, ctx1, ctx2)

    Wtt = jnp.transpose(Wt, (2, 0, 1))
    out = _t4(p1, p2, Wtt, Wb.T, bt.reshape(1, 16), Wfc, bfc.reshape(1, 16),
              Ws, bs.reshape(1, 1))
    return out


# speedup vs baseline: 11.3073x; 1.1125x over previous
import functools

import jax
import jax.numpy as jnp
from jax import lax
from jax.experimental import pallas as pl
from jax.experimental.pallas import tpu as pltpu
from jax.experimental.pallas import tpu_sc as plsc

NN = 50000
EE = 800000
NPAD = 51200
EPAD = 819200
ECH = EPAD // 128
RPT = NPAD // 16
SCH = 8
BLK = 2000
NBLK = NN // BLK
F32 = jnp.float32

@functools.cache
def _mesh():
    return plsc.VectorSubcoreMesh(core_axis_name="c", subcore_axis_name="s",
                                  num_cores=2, num_subcores=16)


def _zero_fill(buf, width):
    z16 = jnp.zeros((16,), F32)

    def body(i, _):
        for h in range(width // 16):
            buf[i, pl.ds(16 * h, 16)] = z16
        return 0

    lax.fori_loop(0, buf.shape[0], body, 0)


def _unit_scatter(src2d, dst2d, table, out, acc, sbuf, dbuf, rows,
                  gsem, tile, chunk0, ntile_chunks):
    r0 = tile * RPT

    _zero_fill(rows[0], rows[0].shape[1])

    def zb(i, _):
        pltpu.sync_copy(rows[0], acc.at[pl.ds(r0 + i * 128, 128)])
        return 0

    lax.fori_loop(0, RPT // 128, zb, 0)
    plsc.subcore_barrier()

    nsuper = ntile_chunks // SCH

    def sbody(si, _):
        cb = chunk0 + si * SCH
        pltpu.sync_copy(src2d.at[pl.ds(cb, SCH)], sbuf)
        pltpu.sync_copy(dst2d.at[pl.ds(cb, SCH)], dbuf)
        pend = pltpu.async_copy(table.at[sbuf.at[0]], rows[0], gsem)
        for j in range(SCH):
            pend.wait()
            if j + 1 < SCH:
                pend = pltpu.async_copy(table.at[sbuf.at[j + 1]],
                                        rows[(j + 1) % 2], gsem)
            pltpu.sync_copy(rows[j % 2], acc.at[dbuf.at[j]], add=True)
        return 0

    lax.fori_loop(0, nsuper, sbody, 0)
    plsc.subcore_barrier()

    def wb(i, _):
        pltpu.sync_copy(acc.at[pl.ds(r0 + i * 128, 128)], rows[0])
        pltpu.sync_copy(rows[0], out.at[pl.ds(r0 + i * 128, 128)])
        return 0

    lax.fori_loop(0, RPT // 128, wb, 0)


def _unit_deg(dst2d, out, acc, dbuf, obuf, tile, chunk0, ntile_chunks):
    r0 = tile * RPT

    def zb(i, _):
        pltpu.sync_copy(obuf, acc.at[pl.ds(r0 + i * 128, 128)])
        return 0

    lax.fori_loop(0, RPT // 128, zb, 0)
    plsc.subcore_barrier()

    one16 = jnp.ones((16,), F32)

    def ob(i, _):
        obuf[i, pl.ds(0, 16)] = one16
        return 0

    lax.fori_loop(0, 128, ob, 0)

    nsuper = ntile_chunks // SCH

    def sbody(si, _):
        cb = chunk0 + si * SCH
        pltpu.sync_copy(dst2d.at[pl.ds(cb, SCH)], dbuf)
        for j in range(SCH):
            pltpu.sync_copy(obuf, acc.at[dbuf.at[j]], add=True)
        return 0

    lax.fori_loop(0, nsuper, sbody, 0)
    plsc.subcore_barrier()

    def wb(i, _):
        pltpu.sync_copy(acc.at[pl.ds(r0 + i * 128, 128)], obuf)
        pltpu.sync_copy(obuf, out.at[pl.ds(r0 + i * 128, 128)])
        return 0

    lax.fori_loop(0, RPT // 128, wb, 0)


@functools.cache
def _make_deg():
    @functools.partial(
        pl.kernel,
        out_type=[jax.ShapeDtypeStruct((NPAD, 16), F32)] * 2,
        mesh=_mesh(),
        compiler_params=pltpu.CompilerParams(use_tc_tiling_on_sc=False),
        scratch_types=[
            pltpu.VMEM_SHARED((NPAD + 16, 16), F32),
            pltpu.VMEM((SCH, 128), jnp.int32),
            pltpu.VMEM((128, 16), F32),
        ],
    )
    def _deg_kernel(dst1, dst2, deg1, deg2, acc, dbuf, obuf):
        c = lax.axis_index("c")
        s = lax.axis_index("s")
        _zero_fill(obuf, 16)

        @pl.when(c == 0)
        def _():
            _unit_deg(dst1, deg1, acc, dbuf, obuf, s, s * 400, 400)

        @pl.when(c == 1)
        def _():
            _unit_deg(dst2, deg2, acc, dbuf, obuf, s, s * 400, 400)

    return _deg_kernel


@functools.cache
def _make_layer12():
    @functools.partial(
        pl.kernel,
        out_type=[jax.ShapeDtypeStruct((NPAD, 32), F32)] * 4,
        mesh=_mesh(),
        compiler_params=pltpu.CompilerParams(use_tc_tiling_on_sc=False),
        scratch_types=[
            pltpu.VMEM_SHARED((NPAD + 16, 32), F32),
            pltpu.VMEM((SCH, 128), jnp.int32),
            pltpu.VMEM((SCH, 128), jnp.int32),
            pltpu.VMEM((128, 32), F32),
            pltpu.VMEM((128, 32), F32),
            pltpu.SemaphoreType.DMA,
        ],
    )
    def _k(src1, dst1, src2, dst2, g1h0, g1h1, g2h0, g2h1,
           o1h0, o1h1, o2h0, o2h1,
           acc, sbuf, dbuf, rows0, rows1, gsem):
        c = lax.axis_index("c")
        s = lax.axis_index("s")
        rows = (rows0, rows1)

        @pl.when(c == 0)
        def _():
            _unit_scatter(src1, dst1, g1h0, o1h0, acc, sbuf, dbuf, rows,
                          gsem, s, s * 400, 400)
            _unit_scatter(src2, dst2, g2h0, o2h0, acc, sbuf, dbuf, rows,
                          gsem, s, s * 400, 400)

        @pl.when(c == 1)
        def _():
            _unit_scatter(src1, dst1, g1h1, o1h1, acc, sbuf, dbuf, rows,
                          gsem, s, s * 400, 400)
            _unit_scatter(src2, dst2, g2h1, o2h1, acc, sbuf, dbuf, rows,
                          gsem, s, s * 400, 400)

    return _k


@functools.cache
def _make_layer3():
    @functools.partial(
        pl.kernel,
        out_type=[jax.ShapeDtypeStruct((NPAD, 32), F32)] * 4,
        mesh=_mesh(),
        compiler_params=pltpu.CompilerParams(use_tc_tiling_on_sc=False),
        scratch_types=[
            pltpu.VMEM_SHARED((NPAD + 16, 32), F32),
            pltpu.VMEM((SCH, 128), jnp.int32),
            pltpu.VMEM((SCH, 128), jnp.int32),
            pltpu.VMEM((128, 32), F32),
            pltpu.VMEM((128, 32), F32),
            pltpu.SemaphoreType.DMA,
        ],
    )
    def _layer3_kernel(src1, dst1, src2, dst2, g3a, g3b,
                       pa0, pa1, pb0, pb1,
                       acc, sbuf, dbuf, rows0, rows1, gsem):
        c = lax.axis_index("c")
        s = lax.axis_index("s")
        rows = (rows0, rows1)

        @pl.when(c == 0)
        def _():
            _unit_scatter(src1, dst1, g3a, pa0, acc, sbuf, dbuf, rows,
                          gsem, s, s * 200, 200)
            _unit_scatter(src2, dst2, g3b, pb0, acc, sbuf, dbuf, rows,
                          gsem, s, s * 200, 200)

        @pl.when(c == 1)
        def _():
            _unit_scatter(src1, dst1, g3a, pa1, acc, sbuf, dbuf, rows,
                          gsem, s, 3200 + s * 200, 200)
            _unit_scatter(src2, dst2, g3b, pb1, acc, sbuf, dbuf, rows,
                          gsem, s, 3200 + s * 200, 200)

    return _layer3_kernel


def _mm1_body(x_ref, w_ref, deg_ref, o0_ref, o1_ref):
    dinv = lax.rsqrt(deg_ref[:, 0:1] + 1.0)
    h = jnp.dot(x_ref[...], w_ref[...], preferred_element_type=F32) * dinv
    o0_ref[...] = h[:, :32]
    o1_ref[...] = h[:, 32:]


def _mm1(x, W, deg):
    return pl.pallas_call(
        _mm1_body,
        grid=(NBLK,),
        in_specs=[
            pl.BlockSpec((BLK, 128), lambda i: (i, 0)),
            pl.BlockSpec((128, 64), lambda i: (0, 0)),
            pl.BlockSpec((BLK, 16), lambda i: (i, 0)),
        ],
        out_specs=[pl.BlockSpec((BLK, 32), lambda i: (i, 0))] * 2,
        out_shape=[jax.ShapeDtypeStruct((NN, 32), F32)] * 2,
    )(x, W, deg)


def _mm_mid_body2(s0, s1, g0, g1, deg, b, w, o0, o1):
    dinv = lax.rsqrt(deg[:, 0:1] + 1.0)
    prev = jnp.concatenate([s0[...] + g0[...], s1[...] + g1[...]], axis=1)
    x = jnp.maximum(prev * dinv + b[...], 0.0)
    h = jnp.dot(x, w[...], preferred_element_type=F32) * dinv
    o0[...] = h[:, :32]
    o1[...] = h[:, 32:]


def _mm_mid_body1(s0, s1, g0, g1, deg, b, w, o0):
    dinv = lax.rsqrt(deg[:, 0:1] + 1.0)
    prev = jnp.concatenate([s0[...] + g0[...], s1[...] + g1[...]], axis=1)
    x = jnp.maximum(prev * dinv + b[...], 0.0)
    o0[...] = jnp.dot(x, w[...], preferred_element_type=F32) * dinv


def _mm_mid(s0, s1, g0, g1, deg, b_row, W, halves):
    fout = W.shape[1]
    nouts = 2 if halves else 1
    return pl.pallas_call(
        _mm_mid_body2 if halves else _mm_mid_body1,
        grid=(NBLK,),
        in_specs=[
            pl.BlockSpec((BLK, 32), lambda i: (i, 0)),
            pl.BlockSpec((BLK, 32), lambda i: (i, 0)),
            pl.BlockSpec((BLK, 32), lambda i: (i, 0)),
            pl.BlockSpec((BLK, 32), lambda i: (i, 0)),
            pl.BlockSpec((BLK, 16), lambda i: (i, 0)),
            pl.BlockSpec((1, 64), lambda i: (0, 0)),
            pl.BlockSpec((64, fout), lambda i: (0, 0)),
        ],
        out_specs=[pl.BlockSpec((BLK, fout // nouts), lambda i: (i, 0))] * nouts,
        out_shape=[jax.ShapeDtypeStruct((NN, fout // nouts), F32)] * nouts,
    )(s0, s1, g0, g1, deg, b_row, W)


def _t1_body(p10, p11, g31, deg1, p20, p21, g32, deg2, b3, wm1, wm2,
             a1_ref, a2_ref, m1_ref, m2_ref, acc1, acc2):
    i = pl.program_id(0)

    @pl.when(i == 0)
    def _():
        acc1[...] = jnp.zeros_like(acc1)
        acc2[...] = jnp.zeros_like(acc2)

    dinv1 = lax.rsqrt(deg1[:, 0:1] + 1.0)
    a1 = (p10[...] + p11[...] + g31[...]) * dinv1 + b3[...]
    dinv2 = lax.rsqrt(deg2[:, 0:1] + 1.0)
    a2 = (p20[...] + p21[...] + g32[...]) * dinv2 + b3[...]
    a1_ref[...] = a1
    a2_ref[...] = a2
    acc1[...] += jnp.sum(a1, axis=0, keepdims=True)
    acc2[...] += jnp.sum(a2, axis=0, keepdims=True)
    inv_n = 1.0 / NN
    m1_ref[...] = jnp.tanh(
        jnp.dot(acc1[...] * inv_n, wm1[...], preferred_element_type=F32))
    m2_ref[...] = jnp.tanh(
        jnp.dot(acc2[...] * inv_n, wm2[...], preferred_element_type=F32))


def _t1(p10, p11, g31, deg1, p20, p21, g32, deg2, b3_row, Wm1, Wm2):
    blk = pl.BlockSpec((BLK, 32), lambda i: (i, 0))
    small = pl.BlockSpec((1, 32), lambda i: (0, 0))
    return pl.pallas_call(
        _t1_body,
        grid=(NBLK,),
        in_specs=[blk, blk, blk, pl.BlockSpec((BLK, 16), lambda i: (i, 0)),
                  blk, blk, blk, pl.BlockSpec((BLK, 16), lambda i: (i, 0)),
                  small,
                  pl.BlockSpec((32, 32), lambda i: (0, 0)),
                  pl.BlockSpec((32, 32), lambda i: (0, 0))],
        out_specs=[blk, blk, small, small],
        out_shape=[jax.ShapeDtypeStruct((NN, 32), F32),
                   jax.ShapeDtypeStruct((NN, 32), F32),
                   jax.ShapeDtypeStruct((1, 32), F32),
                   jax.ShapeDtypeStruct((1, 32), F32)],
        scratch_shapes=[pltpu.VMEM((1, 32), F32), pltpu.VMEM((1, 32), F32)],
    )(p10, p11, g31, deg1, p20, p21, g32, deg2, b3_row, Wm1, Wm2)


def _t2_body(a1, a2, m1, m2, wa, ctx1_ref, ctx2_ref, acc1, acc2):
    i = pl.program_id(0)

    @pl.when(i == 0)
    def _():
        acc1[...] = jnp.zeros_like(acc1)
        acc2[...] = jnp.zeros_like(acc2)

    x1 = jnp.abs(a1[...] - m2[...])
    x2 = jnp.abs(a2[...] - m1[...])
    acc1[...] += jnp.sum(x1, axis=0, keepdims=True)
    acc2[...] += jnp.sum(x2, axis=0, keepdims=True)
    inv_n = 1.0 / NN
    ctx1_ref[...] = jnp.tanh(
        jnp.dot(acc1[...] * inv_n, wa[...], preferred_element_type=F32))
    ctx2_ref[...] = jnp.tanh(
        jnp.dot(acc2[...] * inv_n, wa[...], preferred_element_type=F32))


def _t2(a1, a2, m1, m2, Wa):
    blk = pl.BlockSpec((BLK, 32), lambda i: (i, 0))
    small = pl.BlockSpec((1, 32), lambda i: (0, 0))
    return pl.pallas_call(
        _t2_body,
        grid=(NBLK,),
        in_specs=[blk, blk, small, small,
                  pl.BlockSpec((32, 32), lambda i: (0, 0))],
        out_specs=[small, small],
        out_shape=[jax.ShapeDtypeStruct((1, 32), F32)] * 2,
        scratch_shapes=[pltpu.VMEM((1, 32), F32), pltpu.VMEM((1, 32), F32)],
    )(a1, a2, m1, m2, Wa)


def _t3_body(a1, a2, m1, m2, ctx1, ctx2, p1_ref, p2_ref, acc1, acc2):
    i = pl.program_id(0)

    @pl.when(i == 0)
    def _():
        acc1[...] = jnp.zeros_like(acc1)
        acc2[...] = jnp.zeros_like(acc2)

    x1 = jnp.abs(a1[...] - m2[...])
    x2 = jnp.abs(a2[...] - m1[...])
    s1 = jax.nn.sigmoid(jnp.sum(x1 * ctx1[...], axis=1, keepdims=True))
    s2 = jax.nn.sigmoid(jnp.sum(x2 * ctx2[...], axis=1, keepdims=True))
    acc1[...] += jnp.sum(x1 * s1, axis=0, keepdims=True)
    acc2[...] += jnp.sum(x2 * s2, axis=0, keepdims=True)
    p1_ref[...] = acc1[...]
    p2_ref[...] = acc2[...]


def _t3(a1, a2, m1, m2, ctx1, ctx2):
    blk = pl.BlockSpec((BLK, 32), lambda i: (i, 0))
    small = pl.BlockSpec((1, 32), lambda i: (0, 0))
    return pl.pallas_call(
        _t3_body,
        grid=(NBLK,),
        in_specs=[blk, blk, small, small, small, small],
        out_specs=[small, small],
        out_shape=[jax.ShapeDtypeStruct((1, 32), F32)] * 2,
        scratch_shapes=[pltpu.VMEM((1, 32), F32), pltpu.VMEM((1, 32), F32)],
    )(a1, a2, m1, m2, ctx1, ctx2)


def _t4_body(p1, p2, wtt, wbt, btr, wfc, bfcr, ws, bsr, out_ref, sc):
    t = pl.program_id(0)
    a = wtt[0]
    v = jnp.dot(p1[...], a, preferred_element_type=F32)
    s_t = jnp.sum(v * p2[...])
    iota = lax.broadcasted_iota(jnp.int32, (1, 16), 1)
    base = jnp.where(t == 0, jnp.zeros_like(sc[...]), sc[...])
    sc[...] = jnp.where(iota == t, s_t, base)
    combined = jnp.concatenate([p1[...], p2[...]], axis=1)
    block = jnp.dot(combined, wbt[...], preferred_element_type=F32)
    scores = jnp.maximum(sc[...] + block + btr[...], 0.0)
    h = jnp.maximum(
        jnp.dot(scores, wfc[...], preferred_element_type=F32) + bfcr[...], 0.0)
    out_ref[...] = jax.nn.sigmoid(
        jnp.dot(h, ws[...], preferred_element_type=F32) + bsr[...])


def _t4(p1, p2, Wtt, Wbt, btr, Wfc, bfcr, Ws, bsr):
    small = pl.BlockSpec((1, 32), lambda t: (0, 0))
    return pl.pallas_call(
        _t4_body,
        grid=(16,),
        in_specs=[small, small,
                  pl.BlockSpec((1, 32, 32), lambda t: (t, 0, 0)),
                  pl.BlockSpec((64, 16), lambda t: (0, 0)),
                  pl.BlockSpec((1, 16), lambda t: (0, 0)),
                  pl.BlockSpec((16, 16), lambda t: (0, 0)),
                  pl.BlockSpec((1, 16), lambda t: (0, 0)),
                  pl.BlockSpec((16, 1), lambda t: (0, 0)),
                  pl.BlockSpec((1, 1), lambda t: (0, 0))],
        out_specs=pl.BlockSpec((1, 1), lambda t: (0, 0)),
        out_shape=jax.ShapeDtypeStruct((1, 1), F32),
        scratch_shapes=[pltpu.VMEM((1, 16), F32)],
    )(p1, p2, Wtt, Wbt, btr, Wfc, bfcr, Ws, bsr)


def _pad_edges(ei):
    pad = EPAD - EE
    src = jnp.concatenate([ei[0], jnp.zeros((pad,), jnp.int32)])
    dst = jnp.concatenate([ei[1], jnp.full((pad,), NPAD, jnp.int32)])
    return src.reshape(ECH, 128), dst.reshape(ECH, 128)


def kernel(features_1, edge_index_1, features_2, edge_index_2, W1, b1, W2, b2,
           W3, b3, Wm1, Wm2, Wa, Wt, Wb, bt, Wfc, bfc, Ws, bs):
    src1, dst1 = _pad_edges(edge_index_1)
    src2, dst2 = _pad_edges(edge_index_2)

    deg1, deg2 = _make_deg()(dst1, dst2)

    b1r = b1.reshape(1, 64)
    b2r = b2.reshape(1, 64)
    b3r = b3.reshape(1, 32)

    g1h0, g1h1 = _mm1(features_1, W1, deg1)
    g2h0, g2h1 = _mm1(features_2, W1, deg2)
    s1h0, s1h1, s2h0, s2h1 = _make_layer12()(src1, dst1, src2, dst2,
                                             g1h0, g1h1, g2h0, g2h1)

    q1h0, q1h1 = _mm_mid(s1h0, s1h1, g1h0, g1h1, deg1, b1r, W2, True)
    q2h0, q2h1 = _mm_mid(s2h0, s2h1, g2h0, g2h1, deg2, b1r, W2, True)
    t1h0, t1h1, t2h0, t2h1 = _make_layer12()(src1, dst1, src2, dst2,
                                             q1h0, q1h1, q2h0, q2h1)

    (g31,) = _mm_mid(t1h0, t1h1, q1h0, q1h1, deg1, b2r, W3, False)
    (g32,) = _mm_mid(t2h0, t2h1, q2h0, q2h1, deg2, b2r, W3, False)
    pa0, pa1, pb0, pb1 = _make_layer3()(src1, dst1, src2, dst2, g31, g32)

    a1, a2, m1, m2 = _t1(pa0, pa1, g31, deg1, pb0, pb1, g32, deg2,
                         b3r, Wm1, Wm2)
    ctx1, ctx2 = _t2(a1, a2, m1, m2, Wa)
    p1, p2 = _t3(a1, a2, m1, m2, ctx1, ctx2)

    Wtt = jnp.transpose(Wt, (2, 0, 1))
    out = _t4(p1, p2, Wtt, Wb.T, bt.reshape(1, 16), Wfc, bfc.reshape(1, 16),
              Ws, bs.reshape(1, 1))
    return out


# speedup vs baseline: 13.3066x; 1.1768x over previous
import functools

import jax
import jax.numpy as jnp
from jax import lax
from jax.experimental import pallas as pl
from jax.experimental.pallas import tpu as pltpu
from jax.experimental.pallas import tpu_sc as plsc

NN = 50000
EE = 800000
NPAD = 51200
EPAD = 819200
ECH = EPAD // 128
RPT = NPAD // 16
SCH = 8
BLK = 2000
NBLK = NN // BLK
F32 = jnp.float32

@functools.cache
def _mesh():
    return plsc.VectorSubcoreMesh(core_axis_name="c", subcore_axis_name="s",
                                  num_cores=2, num_subcores=16)


def _zero_fill(buf, width):
    z16 = jnp.zeros((16,), F32)

    def body(i, _):
        for h in range(width // 16):
            buf[i, pl.ds(16 * h, 16)] = z16
        return 0

    lax.fori_loop(0, buf.shape[0], body, 0)


def _unit_scatter(src2d, dst2d, table, out, acc, sbuf, dbuf, rows,
                  gsem, ssem, tile, chunk0, ntile_chunks):
    r0 = tile * RPT

    _zero_fill(rows[0], rows[0].shape[1])

    def zb(i, _):
        pltpu.sync_copy(rows[0], acc.at[pl.ds(r0 + i * 128, 128)])
        return 0

    lax.fori_loop(0, RPT // 128, zb, 0)
    plsc.subcore_barrier()

    nsuper = ntile_chunks // SCH

    def sbody(si, _):
        cb = chunk0 + si * SCH
        pltpu.sync_copy(src2d.at[pl.ds(cb, SCH)], sbuf)
        pltpu.sync_copy(dst2d.at[pl.ds(cb, SCH)], dbuf)
        gd = [pltpu.async_copy(table.at[sbuf.at[0]], rows[0], gsem),
              pltpu.async_copy(table.at[sbuf.at[1]], rows[1], gsem)]
        sd = []
        for j in range(SCH):
            gd.pop(0).wait()
            sd.append(pltpu.async_copy(rows[j % 4], acc.at[dbuf.at[j]],
                                       ssem, add=True))
            nj = j + 2
            if nj < SCH:
                if j >= 2:
                    sd.pop(0).wait()
                gd.append(pltpu.async_copy(table.at[sbuf.at[nj]],
                                           rows[nj % 4], gsem))
        for s in sd:
            s.wait()
        return 0

    lax.fori_loop(0, nsuper, sbody, 0)
    plsc.subcore_barrier()

    def wb(i, _):
        pltpu.sync_copy(acc.at[pl.ds(r0 + i * 128, 128)], rows[0])
        pltpu.sync_copy(rows[0], out.at[pl.ds(r0 + i * 128, 128)])
        return 0

    lax.fori_loop(0, RPT // 128, wb, 0)


def _unit_deg(dst2d, out, acc, dbuf, obuf, tile, chunk0, ntile_chunks):
    r0 = tile * RPT

    def zb(i, _):
        pltpu.sync_copy(obuf, acc.at[pl.ds(r0 + i * 128, 128)])
        return 0

    lax.fori_loop(0, RPT // 128, zb, 0)
    plsc.subcore_barrier()

    one16 = jnp.ones((16,), F32)

    def ob(i, _):
        obuf[i, pl.ds(0, 16)] = one16
        return 0

    lax.fori_loop(0, 128, ob, 0)

    nsuper = ntile_chunks // SCH

    def sbody(si, _):
        cb = chunk0 + si * SCH
        pltpu.sync_copy(dst2d.at[pl.ds(cb, SCH)], dbuf)
        for j in range(SCH):
            pltpu.sync_copy(obuf, acc.at[dbuf.at[j]], add=True)
        return 0

    lax.fori_loop(0, nsuper, sbody, 0)
    plsc.subcore_barrier()

    def wb(i, _):
        pltpu.sync_copy(acc.at[pl.ds(r0 + i * 128, 128)], obuf)
        pltpu.sync_copy(obuf, out.at[pl.ds(r0 + i * 128, 128)])
        return 0

    lax.fori_loop(0, RPT // 128, wb, 0)


@functools.cache
def _make_deg():
    @functools.partial(
        pl.kernel,
        out_type=[jax.ShapeDtypeStruct((NPAD, 16), F32)] * 2,
        mesh=_mesh(),
        compiler_params=pltpu.CompilerParams(use_tc_tiling_on_sc=False),
        scratch_types=[
            pltpu.VMEM_SHARED((NPAD + 16, 16), F32),
            pltpu.VMEM((SCH, 128), jnp.int32),
            pltpu.VMEM((128, 16), F32),
        ],
    )
    def _deg_kernel(dst1, dst2, deg1, deg2, acc, dbuf, obuf):
        c = lax.axis_index("c")
        s = lax.axis_index("s")
        _zero_fill(obuf, 16)

        @pl.when(c == 0)
        def _():
            _unit_deg(dst1, deg1, acc, dbuf, obuf, s, s * 400, 400)

        @pl.when(c == 1)
        def _():
            _unit_deg(dst2, deg2, acc, dbuf, obuf, s, s * 400, 400)

    return _deg_kernel


@functools.cache
def _make_layer12():
    @functools.partial(
        pl.kernel,
        out_type=[jax.ShapeDtypeStruct((NPAD, 32), F32)] * 4,
        mesh=_mesh(),
        compiler_params=pltpu.CompilerParams(use_tc_tiling_on_sc=False),
        scratch_types=[
            pltpu.VMEM_SHARED((NPAD + 16, 32), F32),
            pltpu.VMEM((SCH, 128), jnp.int32),
            pltpu.VMEM((SCH, 128), jnp.int32),
            pltpu.VMEM((128, 32), F32),
            pltpu.VMEM((128, 32), F32),
            pltpu.VMEM((128, 32), F32),
            pltpu.VMEM((128, 32), F32),
            pltpu.SemaphoreType.DMA,
            pltpu.SemaphoreType.DMA,
        ],
    )
    def _k(src1, dst1, src2, dst2, g1h0, g1h1, g2h0, g2h1,
           o1h0, o1h1, o2h0, o2h1,
           acc, sbuf, dbuf, rows0, rows1, rows2, rows3, gsem, ssem):
        c = lax.axis_index("c")
        s = lax.axis_index("s")
        rows = (rows0, rows1, rows2, rows3)

        @pl.when(c == 0)
        def _():
            _unit_scatter(src1, dst1, g1h0, o1h0, acc, sbuf, dbuf, rows,
                          gsem, ssem, s, s * 400, 400)
            _unit_scatter(src2, dst2, g2h0, o2h0, acc, sbuf, dbuf, rows,
                          gsem, ssem, s, s * 400, 400)

        @pl.when(c == 1)
        def _():
            _unit_scatter(src1, dst1, g1h1, o1h1, acc, sbuf, dbuf, rows,
                          gsem, ssem, s, s * 400, 400)
            _unit_scatter(src2, dst2, g2h1, o2h1, acc, sbuf, dbuf, rows,
                          gsem, ssem, s, s * 400, 400)

    return _k


@functools.cache
def _make_layer3():
    @functools.partial(
        pl.kernel,
        out_type=[jax.ShapeDtypeStruct((NPAD, 32), F32)] * 4,
        mesh=_mesh(),
        compiler_params=pltpu.CompilerParams(use_tc_tiling_on_sc=False),
        scratch_types=[
            pltpu.VMEM_SHARED((NPAD + 16, 32), F32),
            pltpu.VMEM((SCH, 128), jnp.int32),
            pltpu.VMEM((SCH, 128), jnp.int32),
            pltpu.VMEM((128, 32), F32),
            pltpu.VMEM((128, 32), F32),
            pltpu.VMEM((128, 32), F32),
            pltpu.VMEM((128, 32), F32),
            pltpu.SemaphoreType.DMA,
            pltpu.SemaphoreType.DMA,
        ],
    )
    def _layer3_kernel(src1, dst1, src2, dst2, g3a, g3b,
                       pa0, pa1, pb0, pb1,
                       acc, sbuf, dbuf, rows0, rows1, rows2, rows3, gsem, ssem):
        c = lax.axis_index("c")
        s = lax.axis_index("s")
        rows = (rows0, rows1, rows2, rows3)

        @pl.when(c == 0)
        def _():
            _unit_scatter(src1, dst1, g3a, pa0, acc, sbuf, dbuf, rows,
                          gsem, ssem, s, s * 200, 200)
            _unit_scatter(src2, dst2, g3b, pb0, acc, sbuf, dbuf, rows,
                          gsem, ssem, s, s * 200, 200)

        @pl.when(c == 1)
        def _():
            _unit_scatter(src1, dst1, g3a, pa1, acc, sbuf, dbuf, rows,
                          gsem, ssem, s, 3200 + s * 200, 200)
            _unit_scatter(src2, dst2, g3b, pb1, acc, sbuf, dbuf, rows,
                          gsem, ssem, s, 3200 + s * 200, 200)

    return _layer3_kernel


def _mm1_body(x_ref, w_ref, deg_ref, o0_ref, o1_ref):
    dinv = lax.rsqrt(deg_ref[:, 0:1] + 1.0)
    h = jnp.dot(x_ref[...], w_ref[...], preferred_element_type=F32) * dinv
    o0_ref[...] = h[:, :32]
    o1_ref[...] = h[:, 32:]


def _mm1(x, W, deg):
    return pl.pallas_call(
        _mm1_body,
        grid=(NBLK,),
        in_specs=[
            pl.BlockSpec((BLK, 128), lambda i: (i, 0)),
            pl.BlockSpec((128, 64), lambda i: (0, 0)),
            pl.BlockSpec((BLK, 16), lambda i: (i, 0)),
        ],
        out_specs=[pl.BlockSpec((BLK, 32), lambda i: (i, 0))] * 2,
        out_shape=[jax.ShapeDtypeStruct((NN, 32), F32)] * 2,
    )(x, W, deg)


def _mm_mid_body2(s0, s1, g0, g1, deg, b, w, o0, o1):
    dinv = lax.rsqrt(deg[:, 0:1] + 1.0)
    prev = jnp.concatenate([s0[...] + g0[...], s1[...] + g1[...]], axis=1)
    x = jnp.maximum(prev * dinv + b[...], 0.0)
    h = jnp.dot(x, w[...], preferred_element_type=F32) * dinv
    o0[...] = h[:, :32]
    o1[...] = h[:, 32:]


def _mm_mid_body1(s0, s1, g0, g1, deg, b, w, o0):
    dinv = lax.rsqrt(deg[:, 0:1] + 1.0)
    prev = jnp.concatenate([s0[...] + g0[...], s1[...] + g1[...]], axis=1)
    x = jnp.maximum(prev * dinv + b[...], 0.0)
    o0[...] = jnp.dot(x, w[...], preferred_element_type=F32) * dinv


def _mm_mid(s0, s1, g0, g1, deg, b_row, W, halves):
    fout = W.shape[1]
    nouts = 2 if halves else 1
    return pl.pallas_call(
        _mm_mid_body2 if halves else _mm_mid_body1,
        grid=(NBLK,),
        in_specs=[
            pl.BlockSpec((BLK, 32), lambda i: (i, 0)),
            pl.BlockSpec((BLK, 32), lambda i: (i, 0)),
            pl.BlockSpec((BLK, 32), lambda i: (i, 0)),
            pl.BlockSpec((BLK, 32), lambda i: (i, 0)),
            pl.BlockSpec((BLK, 16), lambda i: (i, 0)),
            pl.BlockSpec((1, 64), lambda i: (0, 0)),
            pl.BlockSpec((64, fout), lambda i: (0, 0)),
        ],
        out_specs=[pl.BlockSpec((BLK, fout // nouts), lambda i: (i, 0))] * nouts,
        out_shape=[jax.ShapeDtypeStruct((NN, fout // nouts), F32)] * nouts,
    )(s0, s1, g0, g1, deg, b_row, W)


def _t1_body(p10, p11, g31, deg1, p20, p21, g32, deg2, b3, wm1, wm2,
             a1_ref, a2_ref, m1_ref, m2_ref, acc1, acc2):
    i = pl.program_id(0)

    @pl.when(i == 0)
    def _():
        acc1[...] = jnp.zeros_like(acc1)
        acc2[...] = jnp.zeros_like(acc2)

    dinv1 = lax.rsqrt(deg1[:, 0:1] + 1.0)
    a1 = (p10[...] + p11[...] + g31[...]) * dinv1 + b3[...]
    dinv2 = lax.rsqrt(deg2[:, 0:1] + 1.0)
    a2 = (p20[...] + p21[...] + g32[...]) * dinv2 + b3[...]
    a1_ref[...] = a1
    a2_ref[...] = a2
    acc1[...] += jnp.sum(a1, axis=0, keepdims=True)
    acc2[...] += jnp.sum(a2, axis=0, keepdims=True)
    inv_n = 1.0 / NN
    m1_ref[...] = jnp.tanh(
        jnp.dot(acc1[...] * inv_n, wm1[...], preferred_element_type=F32))
    m2_ref[...] = jnp.tanh(
        jnp.dot(acc2[...] * inv_n, wm2[...], preferred_element_type=F32))


def _t1(p10, p11, g31, deg1, p20, p21, g32, deg2, b3_row, Wm1, Wm2):
    blk = pl.BlockSpec((BLK, 32), lambda i: (i, 0))
    small = pl.BlockSpec((1, 32), lambda i: (0, 0))
    return pl.pallas_call(
        _t1_body,
        grid=(NBLK,),
        in_specs=[blk, blk, blk, pl.BlockSpec((BLK, 16), lambda i: (i, 0)),
                  blk, blk, blk, pl.BlockSpec((BLK, 16), lambda i: (i, 0)),
                  small,
                  pl.BlockSpec((32, 32), lambda i: (0, 0)),
                  pl.BlockSpec((32, 32), lambda i: (0, 0))],
        out_specs=[blk, blk, small, small],
        out_shape=[jax.ShapeDtypeStruct((NN, 32), F32),
                   jax.ShapeDtypeStruct((NN, 32), F32),
                   jax.ShapeDtypeStruct((1, 32), F32),
                   jax.ShapeDtypeStruct((1, 32), F32)],
        scratch_shapes=[pltpu.VMEM((1, 32), F32), pltpu.VMEM((1, 32), F32)],
    )(p10, p11, g31, deg1, p20, p21, g32, deg2, b3_row, Wm1, Wm2)


def _t2_body(a1, a2, m1, m2, wa, ctx1_ref, ctx2_ref, acc1, acc2):
    i = pl.program_id(0)

    @pl.when(i == 0)
    def _():
        acc1[...] = jnp.zeros_like(acc1)
        acc2[...] = jnp.zeros_like(acc2)

    x1 = jnp.abs(a1[...] - m2[...])
    x2 = jnp.abs(a2[...] - m1[...])
    acc1[...] += jnp.sum(x1, axis=0, keepdims=True)
    acc2[...] += jnp.sum(x2, axis=0, keepdims=True)
    inv_n = 1.0 / NN
    ctx1_ref[...] = jnp.tanh(
        jnp.dot(acc1[...] * inv_n, wa[...], preferred_element_type=F32))
    ctx2_ref[...] = jnp.tanh(
        jnp.dot(acc2[...] * inv_n, wa[...], preferred_element_type=F32))


def _t2(a1, a2, m1, m2, Wa):
    blk = pl.BlockSpec((BLK, 32), lambda i: (i, 0))
    small = pl.BlockSpec((1, 32), lambda i: (0, 0))
    return pl.pallas_call(
        _t2_body,
        grid=(NBLK,),
        in_specs=[blk, blk, small, small,
                  pl.BlockSpec((32, 32), lambda i: (0, 0))],
        out_specs=[small, small],
        out_shape=[jax.ShapeDtypeStruct((1, 32), F32)] * 2,
        scratch_shapes=[pltpu.VMEM((1, 32), F32), pltpu.VMEM((1, 32), F32)],
    )(a1, a2, m1, m2, Wa)


def _t3_body(a1, a2, m1, m2, ctx1, ctx2, p1_ref, p2_ref, acc1, acc2):
    i = pl.program_id(0)

    @pl.when(i == 0)
    def _():
        acc1[...] = jnp.zeros_like(acc1)
        acc2[...] = jnp.zeros_like(acc2)

    x1 = jnp.abs(a1[...] - m2[...])
    x2 = jnp.abs(a2[...] - m1[...])
    s1 = jax.nn.sigmoid(jnp.sum(x1 * ctx1[...], axis=1, keepdims=True))
    s2 = jax.nn.sigmoid(jnp.sum(x2 * ctx2[...], axis=1, keepdims=True))
    acc1[...] += jnp.sum(x1 * s1, axis=0, keepdims=True)
    acc2[...] += jnp.sum(x2 * s2, axis=0, keepdims=True)
    p1_ref[...] = acc1[...]
    p2_ref[...] = acc2[...]


def _t3(a1, a2, m1, m2, ctx1, ctx2):
    blk = pl.BlockSpec((BLK, 32), lambda i: (i, 0))
    small = pl.BlockSpec((1, 32), lambda i: (0, 0))
    return pl.pallas_call(
        _t3_body,
        grid=(NBLK,),
        in_specs=[blk, blk, small, small, small, small],
        out_specs=[small, small],
        out_shape=[jax.ShapeDtypeStruct((1, 32), F32)] * 2,
        scratch_shapes=[pltpu.VMEM((1, 32), F32), pltpu.VMEM((1, 32), F32)],
    )(a1, a2, m1, m2, ctx1, ctx2)


def _t4_body(p1, p2, wtt, wbt, btr, wfc, bfcr, ws, bsr, out_ref, sc):
    t = pl.program_id(0)
    a = wtt[0]
    v = jnp.dot(p1[...], a, preferred_element_type=F32)
    s_t = jnp.sum(v * p2[...])
    iota = lax.broadcasted_iota(jnp.int32, (1, 16), 1)
    base = jnp.where(t == 0, jnp.zeros_like(sc[...]), sc[...])
    sc[...] = jnp.where(iota == t, s_t, base)
    combined = jnp.concatenate([p1[...], p2[...]], axis=1)
    block = jnp.dot(combined, wbt[...], preferred_element_type=F32)
    scores = jnp.maximum(sc[...] + block + btr[...], 0.0)
    h = jnp.maximum(
        jnp.dot(scores, wfc[...], preferred_element_type=F32) + bfcr[...], 0.0)
    out_ref[...] = jax.nn.sigmoid(
        jnp.dot(h, ws[...], preferred_element_type=F32) + bsr[...])


def _t4(p1, p2, Wtt, Wbt, btr, Wfc, bfcr, Ws, bsr):
    small = pl.BlockSpec((1, 32), lambda t: (0, 0))
    return pl.pallas_call(
        _t4_body,
        grid=(16,),
        in_specs=[small, small,
                  pl.BlockSpec((1, 32, 32), lambda t: (t, 0, 0)),
                  pl.BlockSpec((64, 16), lambda t: (0, 0)),
                  pl.BlockSpec((1, 16), lambda t: (0, 0)),
                  pl.BlockSpec((16, 16), lambda t: (0, 0)),
                  pl.BlockSpec((1, 16), lambda t: (0, 0)),
                  pl.BlockSpec((16, 1), lambda t: (0, 0)),
                  pl.BlockSpec((1, 1), lambda t: (0, 0))],
        out_specs=pl.BlockSpec((1, 1), lambda t: (0, 0)),
        out_shape=jax.ShapeDtypeStruct((1, 1), F32),
        scratch_shapes=[pltpu.VMEM((1, 16), F32)],
    )(p1, p2, Wtt, Wbt, btr, Wfc, bfcr, Ws, bsr)


def _pad_edges(ei):
    pad = EPAD - EE
    src = jnp.concatenate([ei[0], jnp.zeros((pad,), jnp.int32)])
    dst = jnp.concatenate([ei[1], jnp.full((pad,), NPAD, jnp.int32)])
    return src.reshape(ECH, 128), dst.reshape(ECH, 128)


def kernel(features_1, edge_index_1, features_2, edge_index_2, W1, b1, W2, b2,
           W3, b3, Wm1, Wm2, Wa, Wt, Wb, bt, Wfc, bfc, Ws, bs):
    src1, dst1 = _pad_edges(edge_index_1)
    src2, dst2 = _pad_edges(edge_index_2)

    deg1, deg2 = _make_deg()(dst1, dst2)

    b1r = b1.reshape(1, 64)
    b2r = b2.reshape(1, 64)
    b3r = b3.reshape(1, 32)

    g1h0, g1h1 = _mm1(features_1, W1, deg1)
    g2h0, g2h1 = _mm1(features_2, W1, deg2)
    s1h0, s1h1, s2h0, s2h1 = _make_layer12()(src1, dst1, src2, dst2,
                                             g1h0, g1h1, g2h0, g2h1)

    q1h0, q1h1 = _mm_mid(s1h0, s1h1, g1h0, g1h1, deg1, b1r, W2, True)
    q2h0, q2h1 = _mm_mid(s2h0, s2h1, g2h0, g2h1, deg2, b1r, W2, True)
    t1h0, t1h1, t2h0, t2h1 = _make_layer12()(src1, dst1, src2, dst2,
                                             q1h0, q1h1, q2h0, q2h1)

    (g31,) = _mm_mid(t1h0, t1h1, q1h0, q1h1, deg1, b2r, W3, False)
    (g32,) = _mm_mid(t2h0, t2h1, q2h0, q2h1, deg2, b2r, W3, False)
    pa0, pa1, pb0, pb1 = _make_layer3()(src1, dst1, src2, dst2, g31, g32)

    a1, a2, m1, m2 = _t1(pa0, pa1, g31, deg1, pb0, pb1, g32, deg2,
                         b3r, Wm1, Wm2)
    ctx1, ctx2 = _t2(a1, a2, m1, m2, Wa)
    p1, p2 = _t3(a1, a2, m1, m2, ctx1, ctx2)

    Wtt = jnp.transpose(Wt, (2, 0, 1))
    out = _t4(p1, p2, Wtt, Wb.T, bt.reshape(1, 16), Wfc, bfc.reshape(1, 16),
              Ws, bs.reshape(1, 1))
    return out


# speedup vs baseline: 13.7698x; 1.0348x over previous
import functools

import jax
import jax.numpy as jnp
from jax import lax
from jax.experimental import pallas as pl
from jax.experimental.pallas import tpu as pltpu
from jax.experimental.pallas import tpu_sc as plsc

NN = 50000
EE = 800000
NPAD = 51200
EPAD = 819200
ECH = EPAD // 128
RPT = NPAD // 16
SCH = 8
BLK = 2000
NBLK = NN // BLK
F32 = jnp.float32

@functools.cache
def _mesh():
    return plsc.VectorSubcoreMesh(core_axis_name="c", subcore_axis_name="s",
                                  num_cores=2, num_subcores=16)


def _zero_fill(buf, width):
    z16 = jnp.zeros((16,), F32)

    def body(i, _):
        for h in range(width // 16):
            buf[i, pl.ds(16 * h, 16)] = z16
        return 0

    lax.fori_loop(0, buf.shape[0], body, 0)


def _unit_scatter(src2d, dst2d, table, out, acc, sbuf, dbuf, rows,
                  gsem, ssem, tile, chunk0, ntile_chunks, sch):
    r0 = tile * RPT

    _zero_fill(rows[0], rows[0].shape[1])

    def zb(i, _):
        pltpu.sync_copy(rows[0], acc.at[pl.ds(r0 + i * 128, 128)])
        return 0

    lax.fori_loop(0, RPT // 128, zb, 0)
    plsc.subcore_barrier()

    nsuper = ntile_chunks // sch

    def sbody(si, _):
        cb = chunk0 + si * sch
        pltpu.sync_copy(src2d.at[pl.ds(cb, sch)], sbuf)
        pltpu.sync_copy(dst2d.at[pl.ds(cb, sch)], dbuf)
        gd = [pltpu.async_copy(table.at[sbuf.at[0]], rows[0], gsem),
              pltpu.async_copy(table.at[sbuf.at[1]], rows[1], gsem)]
        sd = []
        for j in range(sch):
            gd.pop(0).wait()
            sd.append(pltpu.async_copy(rows[j % 4], acc.at[dbuf.at[j]],
                                       ssem, add=True))
            nj = j + 2
            if nj < sch:
                if j >= 2:
                    sd.pop(0).wait()
                gd.append(pltpu.async_copy(table.at[sbuf.at[nj]],
                                           rows[nj % 4], gsem))
        for s in sd:
            s.wait()
        return 0

    lax.fori_loop(0, nsuper, sbody, 0)
    plsc.subcore_barrier()

    def wb(i, _):
        pltpu.sync_copy(acc.at[pl.ds(r0 + i * 128, 128)], rows[0])
        pltpu.sync_copy(rows[0], out.at[pl.ds(r0 + i * 128, 128)])
        return 0

    lax.fori_loop(0, RPT // 128, wb, 0)


def _unit_deg(dst2d, out, acc, dbuf, obuf, tile, chunk0, ntile_chunks):
    r0 = tile * RPT

    def zb(i, _):
        pltpu.sync_copy(obuf, acc.at[pl.ds(r0 + i * 128, 128)])
        return 0

    lax.fori_loop(0, RPT // 128, zb, 0)
    plsc.subcore_barrier()

    one16 = jnp.ones((16,), F32)

    def ob(i, _):
        obuf[i, pl.ds(0, 16)] = one16
        return 0

    lax.fori_loop(0, 128, ob, 0)

    nsuper = ntile_chunks // SCH

    def sbody(si, _):
        cb = chunk0 + si * SCH
        pltpu.sync_copy(dst2d.at[pl.ds(cb, SCH)], dbuf)
        for j in range(SCH):
            pltpu.sync_copy(obuf, acc.at[dbuf.at[j]], add=True)
        return 0

    lax.fori_loop(0, nsuper, sbody, 0)
    plsc.subcore_barrier()

    def wb(i, _):
        pltpu.sync_copy(acc.at[pl.ds(r0 + i * 128, 128)], obuf)
        pltpu.sync_copy(obuf, out.at[pl.ds(r0 + i * 128, 128)])
        return 0

    lax.fori_loop(0, RPT // 128, wb, 0)


@functools.cache
def _make_deg():
    @functools.partial(
        pl.kernel,
        out_type=[jax.ShapeDtypeStruct((NPAD, 16), F32)] * 2,
        mesh=_mesh(),
        compiler_params=pltpu.CompilerParams(use_tc_tiling_on_sc=False),
        scratch_types=[
            pltpu.VMEM_SHARED((NPAD + 16, 16), F32),
            pltpu.VMEM((SCH, 128), jnp.int32),
            pltpu.VMEM((128, 16), F32),
        ],
    )
    def _deg_kernel(dst1, dst2, deg1, deg2, acc, dbuf, obuf):
        c = lax.axis_index("c")
        s = lax.axis_index("s")
        _zero_fill(obuf, 16)

        @pl.when(c == 0)
        def _():
            _unit_deg(dst1, deg1, acc, dbuf, obuf, s, s * 400, 400)

        @pl.when(c == 1)
        def _():
            _unit_deg(dst2, deg2, acc, dbuf, obuf, s, s * 400, 400)

    return _deg_kernel


@functools.cache
def _make_layer12():
    @functools.partial(
        pl.kernel,
        out_type=[jax.ShapeDtypeStruct((NPAD, 32), F32)] * 4,
        mesh=_mesh(),
        compiler_params=pltpu.CompilerParams(use_tc_tiling_on_sc=False),
        scratch_types=[
            pltpu.VMEM_SHARED((NPAD + 16, 32), F32),
            pltpu.VMEM((16, 128), jnp.int32),
            pltpu.VMEM((16, 128), jnp.int32),
            pltpu.VMEM((128, 32), F32),
            pltpu.VMEM((128, 32), F32),
            pltpu.VMEM((128, 32), F32),
            pltpu.VMEM((128, 32), F32),
            pltpu.SemaphoreType.DMA,
            pltpu.SemaphoreType.DMA,
        ],
    )
    def _k(src1, dst1, src2, dst2, g1h0, g1h1, g2h0, g2h1,
           o1h0, o1h1, o2h0, o2h1,
           acc, sbuf, dbuf, rows0, rows1, rows2, rows3, gsem, ssem):
        c = lax.axis_index("c")
        s = lax.axis_index("s")
        rows = (rows0, rows1, rows2, rows3)

        @pl.when(c == 0)
        def _():
            _unit_scatter(src1, dst1, g1h0, o1h0, acc, sbuf, dbuf, rows,
                          gsem, ssem, s, s * 400, 400, 16)
            _unit_scatter(src2, dst2, g2h0, o2h0, acc, sbuf, dbuf, rows,
                          gsem, ssem, s, s * 400, 400, 16)

        @pl.when(c == 1)
        def _():
            _unit_scatter(src1, dst1, g1h1, o1h1, acc, sbuf, dbuf, rows,
                          gsem, ssem, s, s * 400, 400, 16)
            _unit_scatter(src2, dst2, g2h1, o2h1, acc, sbuf, dbuf, rows,
                          gsem, ssem, s, s * 400, 400, 16)

    return _k


@functools.cache
def _make_layer3():
    @functools.partial(
        pl.kernel,
        out_type=[jax.ShapeDtypeStruct((NPAD, 32), F32)] * 4,
        mesh=_mesh(),
        compiler_params=pltpu.CompilerParams(use_tc_tiling_on_sc=False),
        scratch_types=[
            pltpu.VMEM_SHARED((NPAD + 16, 32), F32),
            pltpu.VMEM((8, 128), jnp.int32),
            pltpu.VMEM((8, 128), jnp.int32),
            pltpu.VMEM((128, 32), F32),
            pltpu.VMEM((128, 32), F32),
            pltpu.VMEM((128, 32), F32),
            pltpu.VMEM((128, 32), F32),
            pltpu.SemaphoreType.DMA,
            pltpu.SemaphoreType.DMA,
        ],
    )
    def _layer3_kernel(src1, dst1, src2, dst2, g3a, g3b,
                       pa0, pa1, pb0, pb1,
                       acc, sbuf, dbuf, rows0, rows1, rows2, rows3, gsem, ssem):
        c = lax.axis_index("c")
        s = lax.axis_index("s")
        rows = (rows0, rows1, rows2, rows3)

        @pl.when(c == 0)
        def _():
            _unit_scatter(src1, dst1, g3a, pa0, acc, sbuf, dbuf, rows,
                          gsem, ssem, s, s * 200, 200, 8)
            _unit_scatter(src2, dst2, g3b, pb0, acc, sbuf, dbuf, rows,
                          gsem, ssem, s, s * 200, 200, 8)

        @pl.when(c == 1)
        def _():
            _unit_scatter(src1, dst1, g3a, pa1, acc, sbuf, dbuf, rows,
                          gsem, ssem, s, 3200 + s * 200, 200, 8)
            _unit_scatter(src2, dst2, g3b, pb1, acc, sbuf, dbuf, rows,
                          gsem, ssem, s, 3200 + s * 200, 200, 8)

    return _layer3_kernel


def _mm1_body(x_ref, w_ref, deg_ref, o0_ref, o1_ref):
    dinv = lax.rsqrt(deg_ref[:, 0:1] + 1.0)
    h = jnp.dot(x_ref[...], w_ref[...], preferred_element_type=F32) * dinv
    o0_ref[...] = h[:, :32]
    o1_ref[...] = h[:, 32:]


def _mm1(x, W, deg):
    return pl.pallas_call(
        _mm1_body,
        grid=(NBLK,),
        in_specs=[
            pl.BlockSpec((BLK, 128), lambda i: (i, 0)),
            pl.BlockSpec((128, 64), lambda i: (0, 0)),
            pl.BlockSpec((BLK, 16), lambda i: (i, 0)),
        ],
        out_specs=[pl.BlockSpec((BLK, 32), lambda i: (i, 0))] * 2,
        out_shape=[jax.ShapeDtypeStruct((NN, 32), F32)] * 2,
    )(x, W, deg)


def _mm_mid_body2(s0, s1, g0, g1, deg, b, w, o0, o1):
    dinv = lax.rsqrt(deg[:, 0:1] + 1.0)
    prev = jnp.concatenate([s0[...] + g0[...], s1[...] + g1[...]], axis=1)
    x = jnp.maximum(prev * dinv + b[...], 0.0)
    h = jnp.dot(x, w[...], preferred_element_type=F32) * dinv
    o0[...] = h[:, :32]
    o1[...] = h[:, 32:]


def _mm_mid_body1(s0, s1, g0, g1, deg, b, w, o0):
    dinv = lax.rsqrt(deg[:, 0:1] + 1.0)
    prev = jnp.concatenate([s0[...] + g0[...], s1[...] + g1[...]], axis=1)
    x = jnp.maximum(prev * dinv + b[...], 0.0)
    o0[...] = jnp.dot(x, w[...], preferred_element_type=F32) * dinv


def _mm_mid(s0, s1, g0, g1, deg, b_row, W, halves):
    fout = W.shape[1]
    nouts = 2 if halves else 1
    return pl.pallas_call(
        _mm_mid_body2 if halves else _mm_mid_body1,
        grid=(NBLK,),
        in_specs=[
            pl.BlockSpec((BLK, 32), lambda i: (i, 0)),
            pl.BlockSpec((BLK, 32), lambda i: (i, 0)),
            pl.BlockSpec((BLK, 32), lambda i: (i, 0)),
            pl.BlockSpec((BLK, 32), lambda i: (i, 0)),
            pl.BlockSpec((BLK, 16), lambda i: (i, 0)),
            pl.BlockSpec((1, 64), lambda i: (0, 0)),
            pl.BlockSpec((64, fout), lambda i: (0, 0)),
        ],
        out_specs=[pl.BlockSpec((BLK, fout // nouts), lambda i: (i, 0))] * nouts,
        out_shape=[jax.ShapeDtypeStruct((NN, fout // nouts), F32)] * nouts,
    )(s0, s1, g0, g1, deg, b_row, W)


def _t1_body(p10, p11, g31, deg1, p20, p21, g32, deg2, b3, wm1, wm2,
             a1_ref, a2_ref, m1_ref, m2_ref, acc1, acc2):
    i = pl.program_id(0)

    @pl.when(i == 0)
    def _():
        acc1[...] = jnp.zeros_like(acc1)
        acc2[...] = jnp.zeros_like(acc2)

    dinv1 = lax.rsqrt(deg1[:, 0:1] + 1.0)
    a1 = (p10[...] + p11[...] + g31[...]) * dinv1 + b3[...]
    dinv2 = lax.rsqrt(deg2[:, 0:1] + 1.0)
    a2 = (p20[...] + p21[...] + g32[...]) * dinv2 + b3[...]
    a1_ref[...] = a1
    a2_ref[...] = a2
    acc1[...] += jnp.sum(a1, axis=0, keepdims=True)
    acc2[...] += jnp.sum(a2, axis=0, keepdims=True)
    inv_n = 1.0 / NN
    m1_ref[...] = jnp.tanh(
        jnp.dot(acc1[...] * inv_n, wm1[...], preferred_element_type=F32))
    m2_ref[...] = jnp.tanh(
        jnp.dot(acc2[...] * inv_n, wm2[...], preferred_element_type=F32))


def _t1(p10, p11, g31, deg1, p20, p21, g32, deg2, b3_row, Wm1, Wm2):
    blk = pl.BlockSpec((BLK, 32), lambda i: (i, 0))
    small = pl.BlockSpec((1, 32), lambda i: (0, 0))
    return pl.pallas_call(
        _t1_body,
        grid=(NBLK,),
        in_specs=[blk, blk, blk, pl.BlockSpec((BLK, 16), lambda i: (i, 0)),
                  blk, blk, blk, pl.BlockSpec((BLK, 16), lambda i: (i, 0)),
                  small,
                  pl.BlockSpec((32, 32), lambda i: (0, 0)),
                  pl.BlockSpec((32, 32), lambda i: (0, 0))],
        out_specs=[blk, blk, small, small],
        out_shape=[jax.ShapeDtypeStruct((NN, 32), F32),
                   jax.ShapeDtypeStruct((NN, 32), F32),
                   jax.ShapeDtypeStruct((1, 32), F32),
                   jax.ShapeDtypeStruct((1, 32), F32)],
        scratch_shapes=[pltpu.VMEM((1, 32), F32), pltpu.VMEM((1, 32), F32)],
    )(p10, p11, g31, deg1, p20, p21, g32, deg2, b3_row, Wm1, Wm2)


def _t2_body(a1, a2, m1, m2, wa, ctx1_ref, ctx2_ref, acc1, acc2):
    i = pl.program_id(0)

    @pl.when(i == 0)
    def _():
        acc1[...] = jnp.zeros_like(acc1)
        acc2[...] = jnp.zeros_like(acc2)

    x1 = jnp.abs(a1[...] - m2[...])
    x2 = jnp.abs(a2[...] - m1[...])
    acc1[...] += jnp.sum(x1, axis=0, keepdims=True)
    acc2[...] += jnp.sum(x2, axis=0, keepdims=True)
    inv_n = 1.0 / NN
    ctx1_ref[...] = jnp.tanh(
        jnp.dot(acc1[...] * inv_n, wa[...], preferred_element_type=F32))
    ctx2_ref[...] = jnp.tanh(
        jnp.dot(acc2[...] * inv_n, wa[...], preferred_element_type=F32))


def _t2(a1, a2, m1, m2, Wa):
    blk = pl.BlockSpec((BLK, 32), lambda i: (i, 0))
    small = pl.BlockSpec((1, 32), lambda i: (0, 0))
    return pl.pallas_call(
        _t2_body,
        grid=(NBLK,),
        in_specs=[blk, blk, small, small,
                  pl.BlockSpec((32, 32), lambda i: (0, 0))],
        out_specs=[small, small],
        out_shape=[jax.ShapeDtypeStruct((1, 32), F32)] * 2,
        scratch_shapes=[pltpu.VMEM((1, 32), F32), pltpu.VMEM((1, 32), F32)],
    )(a1, a2, m1, m2, Wa)


def _t3_body(a1, a2, m1, m2, ctx1, ctx2, p1_ref, p2_ref, acc1, acc2):
    i = pl.program_id(0)

    @pl.when(i == 0)
    def _():
        acc1[...] = jnp.zeros_like(acc1)
        acc2[...] = jnp.zeros_like(acc2)

    x1 = jnp.abs(a1[...] - m2[...])
    x2 = jnp.abs(a2[...] - m1[...])
    s1 = jax.nn.sigmoid(jnp.sum(x1 * ctx1[...], axis=1, keepdims=True))
    s2 = jax.nn.sigmoid(jnp.sum(x2 * ctx2[...], axis=1, keepdims=True))
    acc1[...] += jnp.sum(x1 * s1, axis=0, keepdims=True)
    acc2[...] += jnp.sum(x2 * s2, axis=0, keepdims=True)
    p1_ref[...] = acc1[...]
    p2_ref[...] = acc2[...]


def _t3(a1, a2, m1, m2, ctx1, ctx2):
    blk = pl.BlockSpec((BLK, 32), lambda i: (i, 0))
    small = pl.BlockSpec((1, 32), lambda i: (0, 0))
    return pl.pallas_call(
        _t3_body,
        grid=(NBLK,),
        in_specs=[blk, blk, small, small, small, small],
        out_specs=[small, small],
        out_shape=[jax.ShapeDtypeStruct((1, 32), F32)] * 2,
        scratch_shapes=[pltpu.VMEM((1, 32), F32), pltpu.VMEM((1, 32), F32)],
    )(a1, a2, m1, m2, ctx1, ctx2)


def _t4_body(p1, p2, wtt, wbt, btr, wfc, bfcr, ws, bsr, out_ref, sc):
    t = pl.program_id(0)
    a = wtt[0]
    v = jnp.dot(p1[...], a, preferred_element_type=F32)
    s_t = jnp.sum(v * p2[...])
    iota = lax.broadcasted_iota(jnp.int32, (1, 16), 1)
    base = jnp.where(t == 0, jnp.zeros_like(sc[...]), sc[...])
    sc[...] = jnp.where(iota == t, s_t, base)
    combined = jnp.concatenate([p1[...], p2[...]], axis=1)
    block = jnp.dot(combined, wbt[...], preferred_element_type=F32)
    scores = jnp.maximum(sc[...] + block + btr[...], 0.0)
    h = jnp.maximum(
        jnp.dot(scores, wfc[...], preferred_element_type=F32) + bfcr[...], 0.0)
    out_ref[...] = jax.nn.sigmoid(
        jnp.dot(h, ws[...], preferred_element_type=F32) + bsr[...])


def _t4(p1, p2, Wtt, Wbt, btr, Wfc, bfcr, Ws, bsr):
    small = pl.BlockSpec((1, 32), lambda t: (0, 0))
    return pl.pallas_call(
        _t4_body,
        grid=(16,),
        in_specs=[small, small,
                  pl.BlockSpec((1, 32, 32), lambda t: (t, 0, 0)),
                  pl.BlockSpec((64, 16), lambda t: (0, 0)),
                  pl.BlockSpec((1, 16), lambda t: (0, 0)),
                  pl.BlockSpec((16, 16), lambda t: (0, 0)),
                  pl.BlockSpec((1, 16), lambda t: (0, 0)),
                  pl.BlockSpec((16, 1), lambda t: (0, 0)),
                  pl.BlockSpec((1, 1), lambda t: (0, 0))],
        out_specs=pl.BlockSpec((1, 1), lambda t: (0, 0)),
        out_shape=jax.ShapeDtypeStruct((1, 1), F32),
        scratch_shapes=[pltpu.VMEM((1, 16), F32)],
    )(p1, p2, Wtt, Wbt, btr, Wfc, bfcr, Ws, bsr)


def _pad_edges(ei):
    pad = EPAD - EE
    src = jnp.concatenate([ei[0], jnp.zeros((pad,), jnp.int32)])
    dst = jnp.concatenate([ei[1], jnp.full((pad,), NPAD, jnp.int32)])
    return src.reshape(ECH, 128), dst.reshape(ECH, 128)


def kernel(features_1, edge_index_1, features_2, edge_index_2, W1, b1, W2, b2,
           W3, b3, Wm1, Wm2, Wa, Wt, Wb, bt, Wfc, bfc, Ws, bs):
    src1, dst1 = _pad_edges(edge_index_1)
    src2, dst2 = _pad_edges(edge_index_2)

    deg1, deg2 = _make_deg()(dst1, dst2)

    b1r = b1.reshape(1, 64)
    b2r = b2.reshape(1, 64)
    b3r = b3.reshape(1, 32)

    g1h0, g1h1 = _mm1(features_1, W1, deg1)
    g2h0, g2h1 = _mm1(features_2, W1, deg2)
    s1h0, s1h1, s2h0, s2h1 = _make_layer12()(src1, dst1, src2, dst2,
                                             g1h0, g1h1, g2h0, g2h1)

    q1h0, q1h1 = _mm_mid(s1h0, s1h1, g1h0, g1h1, deg1, b1r, W2, True)
    q2h0, q2h1 = _mm_mid(s2h0, s2h1, g2h0, g2h1, deg2, b1r, W2, True)
    t1h0, t1h1, t2h0, t2h1 = _make_layer12()(src1, dst1, src2, dst2,
                                             q1h0, q1h1, q2h0, q2h1)

    (g31,) = _mm_mid(t1h0, t1h1, q1h0, q1h1, deg1, b2r, W3, False)
    (g32,) = _mm_mid(t2h0, t2h1, q2h0, q2h1, deg2, b2r, W3, False)
    pa0, pa1, pb0, pb1 = _make_layer3()(src1, dst1, src2, dst2, g31, g32)

    a1, a2, m1, m2 = _t1(pa0, pa1, g31, deg1, pb0, pb1, g32, deg2,
                         b3r, Wm1, Wm2)
    ctx1, ctx2 = _t2(a1, a2, m1, m2, Wa)
    p1, p2 = _t3(a1, a2, m1, m2, ctx1, ctx2)

    Wtt = jnp.transpose(Wt, (2, 0, 1))
    out = _t4(p1, p2, Wtt, Wb.T, bt.reshape(1, 16), Wfc, bfc.reshape(1, 16),
              Ws, bs.reshape(1, 1))
    return out


# speedup vs baseline: 14.2537x; 1.0351x over previous
import functools

import jax
import jax.numpy as jnp
from jax import lax
from jax.experimental import pallas as pl
from jax.experimental.pallas import tpu as pltpu
from jax.experimental.pallas import tpu_sc as plsc

NN = 50000
EE = 800000
NPAD = 51200
EPAD = 819200
ECH = EPAD // 128
RPT = NPAD // 16
SCH = 8
BLK = 2000
NBLK = NN // BLK
F32 = jnp.float32

@functools.cache
def _mesh():
    return plsc.VectorSubcoreMesh(core_axis_name="c", subcore_axis_name="s",
                                  num_cores=2, num_subcores=16)


def _zero_fill(buf, width):
    z16 = jnp.zeros((16,), F32)

    def body(i, _):
        for h in range(width // 16):
            buf[i, pl.ds(16 * h, 16)] = z16
        return 0

    lax.fori_loop(0, buf.shape[0], body, 0)


def _unit_scatter(src2d, dst2d, table, out, acc, sbuf, dbuf, rows,
                  gsem, ssem, tile, chunk0, ntile_chunks, sch):
    r0 = tile * RPT
    nb = len(rows)
    gdepth = nb - 2

    _zero_fill(rows[0], rows[0].shape[1])

    zd = [pltpu.async_copy(rows[0], acc.at[pl.ds(r0 + i * 128, 128)], gsem)
          for i in range(RPT // 128)]
    for d in zd:
        d.wait()
    plsc.subcore_barrier()

    nsuper = ntile_chunks // sch

    def sbody(si, _):
        cb = chunk0 + si * sch
        pltpu.sync_copy(src2d.at[pl.ds(cb, sch)], sbuf)
        pltpu.sync_copy(dst2d.at[pl.ds(cb, sch)], dbuf)
        gd = [pltpu.async_copy(table.at[sbuf.at[k]], rows[k], gsem)
              for k in range(gdepth)]
        sd = []
        for j in range(sch):
            gd.pop(0).wait()
            sd.append(pltpu.async_copy(rows[j % nb], acc.at[dbuf.at[j]],
                                       ssem, add=True))
            nj = j + gdepth
            if nj < sch:
                if j >= 2:
                    sd.pop(0).wait()
                gd.append(pltpu.async_copy(table.at[sbuf.at[nj]],
                                           rows[nj % nb], gsem))
        for s in sd:
            s.wait()
        return 0

    lax.fori_loop(0, nsuper, sbody, 0)
    plsc.subcore_barrier()

    def wb(i, _):
        b = rows[0]
        pltpu.sync_copy(acc.at[pl.ds(r0 + i * 256, 128)], b)
        d = pltpu.async_copy(b, out.at[pl.ds(r0 + i * 256, 128)], gsem)
        b2 = rows[1]
        pltpu.sync_copy(acc.at[pl.ds(r0 + i * 256 + 128, 128)], b2)
        d2 = pltpu.async_copy(b2, out.at[pl.ds(r0 + i * 256 + 128, 128)], gsem)
        d.wait()
        d2.wait()
        return 0

    lax.fori_loop(0, RPT // 256, wb, 0)


def _unit_deg(dst2d, out, acc, dbuf, obuf, tile, chunk0, ntile_chunks):
    r0 = tile * RPT

    def zb(i, _):
        pltpu.sync_copy(obuf, acc.at[pl.ds(r0 + i * 128, 128)])
        return 0

    lax.fori_loop(0, RPT // 128, zb, 0)
    plsc.subcore_barrier()

    one16 = jnp.ones((16,), F32)

    def ob(i, _):
        obuf[i, pl.ds(0, 16)] = one16
        return 0

    lax.fori_loop(0, 128, ob, 0)

    nsuper = ntile_chunks // SCH

    def sbody(si, _):
        cb = chunk0 + si * SCH
        pltpu.sync_copy(dst2d.at[pl.ds(cb, SCH)], dbuf)
        for j in range(SCH):
            pltpu.sync_copy(obuf, acc.at[dbuf.at[j]], add=True)
        return 0

    lax.fori_loop(0, nsuper, sbody, 0)
    plsc.subcore_barrier()

    def wb(i, _):
        pltpu.sync_copy(acc.at[pl.ds(r0 + i * 128, 128)], obuf)
        pltpu.sync_copy(obuf, out.at[pl.ds(r0 + i * 128, 128)])
        return 0

    lax.fori_loop(0, RPT // 128, wb, 0)


@functools.cache
def _make_deg():
    @functools.partial(
        pl.kernel,
        out_type=[jax.ShapeDtypeStruct((NPAD, 16), F32)] * 2,
        mesh=_mesh(),
        compiler_params=pltpu.CompilerParams(use_tc_tiling_on_sc=False),
        scratch_types=[
            pltpu.VMEM_SHARED((NPAD + 16, 16), F32),
            pltpu.VMEM((SCH, 128), jnp.int32),
            pltpu.VMEM((128, 16), F32),
        ],
    )
    def _deg_kernel(dst1, dst2, deg1, deg2, acc, dbuf, obuf):
        c = lax.axis_index("c")
        s = lax.axis_index("s")
        _zero_fill(obuf, 16)

        @pl.when(c == 0)
        def _():
            _unit_deg(dst1, deg1, acc, dbuf, obuf, s, s * 400, 400)

        @pl.when(c == 1)
        def _():
            _unit_deg(dst2, deg2, acc, dbuf, obuf, s, s * 400, 400)

    return _deg_kernel


@functools.cache
def _make_layer12():
    @functools.partial(
        pl.kernel,
        out_type=[jax.ShapeDtypeStruct((NPAD, 32), F32)] * 4,
        mesh=_mesh(),
        compiler_params=pltpu.CompilerParams(use_tc_tiling_on_sc=False),
        scratch_types=[
            pltpu.VMEM_SHARED((NPAD + 16, 32), F32),
            pltpu.VMEM((16, 128), jnp.int32),
            pltpu.VMEM((16, 128), jnp.int32),
            pltpu.VMEM((128, 32), F32),
            pltpu.VMEM((128, 32), F32),
            pltpu.VMEM((128, 32), F32),
            pltpu.VMEM((128, 32), F32),
            pltpu.VMEM((128, 32), F32),
            pltpu.SemaphoreType.DMA,
            pltpu.SemaphoreType.DMA,
        ],
    )
    def _k(src1, dst1, src2, dst2, g1h0, g1h1, g2h0, g2h1,
           o1h0, o1h1, o2h0, o2h1,
           acc, sbuf, dbuf, rows0, rows1, rows2, rows3, rows4, gsem, ssem):
        c = lax.axis_index("c")
        s = lax.axis_index("s")
        rows = (rows0, rows1, rows2, rows3, rows4)

        @pl.when(c == 0)
        def _():
            _unit_scatter(src1, dst1, g1h0, o1h0, acc, sbuf, dbuf, rows,
                          gsem, ssem, s, s * 400, 400, 16)
            _unit_scatter(src2, dst2, g2h0, o2h0, acc, sbuf, dbuf, rows,
                          gsem, ssem, s, s * 400, 400, 16)

        @pl.when(c == 1)
        def _():
            _unit_scatter(src1, dst1, g1h1, o1h1, acc, sbuf, dbuf, rows,
                          gsem, ssem, s, s * 400, 400, 16)
            _unit_scatter(src2, dst2, g2h1, o2h1, acc, sbuf, dbuf, rows,
                          gsem, ssem, s, s * 400, 400, 16)

    return _k


@functools.cache
def _make_layer3():
    @functools.partial(
        pl.kernel,
        out_type=[jax.ShapeDtypeStruct((NPAD, 32), F32)] * 4,
        mesh=_mesh(),
        compiler_params=pltpu.CompilerParams(use_tc_tiling_on_sc=False),
        scratch_types=[
            pltpu.VMEM_SHARED((NPAD + 16, 32), F32),
            pltpu.VMEM((8, 128), jnp.int32),
            pltpu.VMEM((8, 128), jnp.int32),
            pltpu.VMEM((128, 32), F32),
            pltpu.VMEM((128, 32), F32),
            pltpu.VMEM((128, 32), F32),
            pltpu.VMEM((128, 32), F32),
            pltpu.VMEM((128, 32), F32),
            pltpu.SemaphoreType.DMA,
            pltpu.SemaphoreType.DMA,
        ],
    )
    def _layer3_kernel(src1, dst1, src2, dst2, g3a, g3b,
                       pa0, pa1, pb0, pb1,
                       acc, sbuf, dbuf, rows0, rows1, rows2, rows3, rows4, gsem, ssem):
        c = lax.axis_index("c")
        s = lax.axis_index("s")
        rows = (rows0, rows1, rows2, rows3, rows4)

        @pl.when(c == 0)
        def _():
            _unit_scatter(src1, dst1, g3a, pa0, acc, sbuf, dbuf, rows,
                          gsem, ssem, s, s * 200, 200, 8)
            _unit_scatter(src2, dst2, g3b, pb0, acc, sbuf, dbuf, rows,
                          gsem, ssem, s, s * 200, 200, 8)

        @pl.when(c == 1)
        def _():
            _unit_scatter(src1, dst1, g3a, pa1, acc, sbuf, dbuf, rows,
                          gsem, ssem, s, 3200 + s * 200, 200, 8)
            _unit_scatter(src2, dst2, g3b, pb1, acc, sbuf, dbuf, rows,
                          gsem, ssem, s, 3200 + s * 200, 200, 8)

    return _layer3_kernel


def _mm1_body(x_ref, w_ref, deg_ref, o0_ref, o1_ref):
    dinv = lax.rsqrt(deg_ref[:, 0:1] + 1.0)
    h = jnp.dot(x_ref[...], w_ref[...], preferred_element_type=F32) * dinv
    o0_ref[...] = h[:, :32]
    o1_ref[...] = h[:, 32:]


def _mm1(x, W, deg):
    return pl.pallas_call(
        _mm1_body,
        grid=(NBLK,),
        in_specs=[
            pl.BlockSpec((BLK, 128), lambda i: (i, 0)),
            pl.BlockSpec((128, 64), lambda i: (0, 0)),
            pl.BlockSpec((BLK, 16), lambda i: (i, 0)),
        ],
        out_specs=[pl.BlockSpec((BLK, 32), lambda i: (i, 0))] * 2,
        out_shape=[jax.ShapeDtypeStruct((NN, 32), F32)] * 2,
    )(x, W, deg)


def _mm_mid_body2(s0, s1, g0, g1, deg, b, w, o0, o1):
    dinv = lax.rsqrt(deg[:, 0:1] + 1.0)
    prev = jnp.concatenate([s0[...] + g0[...], s1[...] + g1[...]], axis=1)
    x = jnp.maximum(prev * dinv + b[...], 0.0)
    h = jnp.dot(x, w[...], preferred_element_type=F32) * dinv
    o0[...] = h[:, :32]
    o1[...] = h[:, 32:]


def _mm_mid_body1(s0, s1, g0, g1, deg, b, w, o0):
    dinv = lax.rsqrt(deg[:, 0:1] + 1.0)
    prev = jnp.concatenate([s0[...] + g0[...], s1[...] + g1[...]], axis=1)
    x = jnp.maximum(prev * dinv + b[...], 0.0)
    o0[...] = jnp.dot(x, w[...], preferred_element_type=F32) * dinv


def _mm_mid(s0, s1, g0, g1, deg, b_row, W, halves):
    fout = W.shape[1]
    nouts = 2 if halves else 1
    return pl.pallas_call(
        _mm_mid_body2 if halves else _mm_mid_body1,
        grid=(NBLK,),
        in_specs=[
            pl.BlockSpec((BLK, 32), lambda i: (i, 0)),
            pl.BlockSpec((BLK, 32), lambda i: (i, 0)),
            pl.BlockSpec((BLK, 32), lambda i: (i, 0)),
            pl.BlockSpec((BLK, 32), lambda i: (i, 0)),
            pl.BlockSpec((BLK, 16), lambda i: (i, 0)),
            pl.BlockSpec((1, 64), lambda i: (0, 0)),
            pl.BlockSpec((64, fout), lambda i: (0, 0)),
        ],
        out_specs=[pl.BlockSpec((BLK, fout // nouts), lambda i: (i, 0))] * nouts,
        out_shape=[jax.ShapeDtypeStruct((NN, fout // nouts), F32)] * nouts,
    )(s0, s1, g0, g1, deg, b_row, W)


def _t1_body(p10, p11, g31, deg1, p20, p21, g32, deg2, b3, wm1, wm2,
             a1_ref, a2_ref, m1_ref, m2_ref, acc1, acc2):
    i = pl.program_id(0)

    @pl.when(i == 0)
    def _():
        acc1[...] = jnp.zeros_like(acc1)
        acc2[...] = jnp.zeros_like(acc2)

    dinv1 = lax.rsqrt(deg1[:, 0:1] + 1.0)
    a1 = (p10[...] + p11[...] + g31[...]) * dinv1 + b3[...]
    dinv2 = lax.rsqrt(deg2[:, 0:1] + 1.0)
    a2 = (p20[...] + p21[...] + g32[...]) * dinv2 + b3[...]
    a1_ref[...] = a1
    a2_ref[...] = a2
    acc1[...] += jnp.sum(a1, axis=0, keepdims=True)
    acc2[...] += jnp.sum(a2, axis=0, keepdims=True)
    inv_n = 1.0 / NN
    m1_ref[...] = jnp.tanh(
        jnp.dot(acc1[...] * inv_n, wm1[...], preferred_element_type=F32))
    m2_ref[...] = jnp.tanh(
        jnp.dot(acc2[...] * inv_n, wm2[...], preferred_element_type=F32))


def _t1(p10, p11, g31, deg1, p20, p21, g32, deg2, b3_row, Wm1, Wm2):
    blk = pl.BlockSpec((BLK, 32), lambda i: (i, 0))
    small = pl.BlockSpec((1, 32), lambda i: (0, 0))
    return pl.pallas_call(
        _t1_body,
        grid=(NBLK,),
        in_specs=[blk, blk, blk, pl.BlockSpec((BLK, 16), lambda i: (i, 0)),
                  blk, blk, blk, pl.BlockSpec((BLK, 16), lambda i: (i, 0)),
                  small,
                  pl.BlockSpec((32, 32), lambda i: (0, 0)),
                  pl.BlockSpec((32, 32), lambda i: (0, 0))],
        out_specs=[blk, blk, small, small],
        out_shape=[jax.ShapeDtypeStruct((NN, 32), F32),
                   jax.ShapeDtypeStruct((NN, 32), F32),
                   jax.ShapeDtypeStruct((1, 32), F32),
                   jax.ShapeDtypeStruct((1, 32), F32)],
        scratch_shapes=[pltpu.VMEM((1, 32), F32), pltpu.VMEM((1, 32), F32)],
    )(p10, p11, g31, deg1, p20, p21, g32, deg2, b3_row, Wm1, Wm2)


def _t2_body(a1, a2, m1, m2, wa, ctx1_ref, ctx2_ref, acc1, acc2):
    i = pl.program_id(0)

    @pl.when(i == 0)
    def _():
        acc1[...] = jnp.zeros_like(acc1)
        acc2[...] = jnp.zeros_like(acc2)

    x1 = jnp.abs(a1[...] - m2[...])
    x2 = jnp.abs(a2[...] - m1[...])
    acc1[...] += jnp.sum(x1, axis=0, keepdims=True)
    acc2[...] += jnp.sum(x2, axis=0, keepdims=True)
    inv_n = 1.0 / NN
    ctx1_ref[...] = jnp.tanh(
        jnp.dot(acc1[...] * inv_n, wa[...], preferred_element_type=F32))
    ctx2_ref[...] = jnp.tanh(
        jnp.dot(acc2[...] * inv_n, wa[...], preferred_element_type=F32))


def _t2(a1, a2, m1, m2, Wa):
    blk = pl.BlockSpec((BLK, 32), lambda i: (i, 0))
    small = pl.BlockSpec((1, 32), lambda i: (0, 0))
    return pl.pallas_call(
        _t2_body,
        grid=(NBLK,),
        in_specs=[blk, blk, small, small,
                  pl.BlockSpec((32, 32), lambda i: (0, 0))],
        out_specs=[small, small],
        out_shape=[jax.ShapeDtypeStruct((1, 32), F32)] * 2,
        scratch_shapes=[pltpu.VMEM((1, 32), F32), pltpu.VMEM((1, 32), F32)],
    )(a1, a2, m1, m2, Wa)


def _t3_body(a1, a2, m1, m2, ctx1, ctx2, p1_ref, p2_ref, acc1, acc2):
    i = pl.program_id(0)

    @pl.when(i == 0)
    def _():
        acc1[...] = jnp.zeros_like(acc1)
        acc2[...] = jnp.zeros_like(acc2)

    x1 = jnp.abs(a1[...] - m2[...])
    x2 = jnp.abs(a2[...] - m1[...])
    s1 = jax.nn.sigmoid(jnp.sum(x1 * ctx1[...], axis=1, keepdims=True))
    s2 = jax.nn.sigmoid(jnp.sum(x2 * ctx2[...], axis=1, keepdims=True))
    acc1[...] += jnp.sum(x1 * s1, axis=0, keepdims=True)
    acc2[...] += jnp.sum(x2 * s2, axis=0, keepdims=True)
    p1_ref[...] = acc1[...]
    p2_ref[...] = acc2[...]


def _t3(a1, a2, m1, m2, ctx1, ctx2):
    blk = pl.BlockSpec((BLK, 32), lambda i: (i, 0))
    small = pl.BlockSpec((1, 32), lambda i: (0, 0))
    return pl.pallas_call(
        _t3_body,
        grid=(NBLK,),
        in_specs=[blk, blk, small, small, small, small],
        out_specs=[small, small],
        out_shape=[jax.ShapeDtypeStruct((1, 32), F32)] * 2,
        scratch_shapes=[pltpu.VMEM((1, 32), F32), pltpu.VMEM((1, 32), F32)],
    )(a1, a2, m1, m2, ctx1, ctx2)


def _t4_body(p1, p2, wtt, wbt, btr, wfc, bfcr, ws, bsr, out_ref, sc):
    t = pl.program_id(0)
    a = wtt[0]
    v = jnp.dot(p1[...], a, preferred_element_type=F32)
    s_t = jnp.sum(v * p2[...])
    iota = lax.broadcasted_iota(jnp.int32, (1, 16), 1)
    base = jnp.where(t == 0, jnp.zeros_like(sc[...]), sc[...])
    sc[...] = jnp.where(iota == t, s_t, base)
    combined = jnp.concatenate([p1[...], p2[...]], axis=1)
    block = jnp.dot(combined, wbt[...], preferred_element_type=F32)
    scores = jnp.maximum(sc[...] + block + btr[...], 0.0)
    h = jnp.maximum(
        jnp.dot(scores, wfc[...], preferred_element_type=F32) + bfcr[...], 0.0)
    out_ref[...] = jax.nn.sigmoid(
        jnp.dot(h, ws[...], preferred_element_type=F32) + bsr[...])


def _t4(p1, p2, Wtt, Wbt, btr, Wfc, bfcr, Ws, bsr):
    small = pl.BlockSpec((1, 32), lambda t: (0, 0))
    return pl.pallas_call(
        _t4_body,
        grid=(16,),
        in_specs=[small, small,
                  pl.BlockSpec((1, 32, 32), lambda t: (t, 0, 0)),
                  pl.BlockSpec((64, 16), lambda t: (0, 0)),
                  pl.BlockSpec((1, 16), lambda t: (0, 0)),
                  pl.BlockSpec((16, 16), lambda t: (0, 0)),
                  pl.BlockSpec((1, 16), lambda t: (0, 0)),
                  pl.BlockSpec((16, 1), lambda t: (0, 0)),
                  pl.BlockSpec((1, 1), lambda t: (0, 0))],
        out_specs=pl.BlockSpec((1, 1), lambda t: (0, 0)),
        out_shape=jax.ShapeDtypeStruct((1, 1), F32),
        scratch_shapes=[pltpu.VMEM((1, 16), F32)],
    )(p1, p2, Wtt, Wbt, btr, Wfc, bfcr, Ws, bsr)


def _pad_edges(ei):
    pad = EPAD - EE
    src = jnp.concatenate([ei[0], jnp.zeros((pad,), jnp.int32)])
    dst = jnp.concatenate([ei[1], jnp.full((pad,), NPAD, jnp.int32)])
    return src.reshape(ECH, 128), dst.reshape(ECH, 128)


def kernel(features_1, edge_index_1, features_2, edge_index_2, W1, b1, W2, b2,
           W3, b3, Wm1, Wm2, Wa, Wt, Wb, bt, Wfc, bfc, Ws, bs):
    src1, dst1 = _pad_edges(edge_index_1)
    src2, dst2 = _pad_edges(edge_index_2)

    deg1, deg2 = _make_deg()(dst1, dst2)

    b1r = b1.reshape(1, 64)
    b2r = b2.reshape(1, 64)
    b3r = b3.reshape(1, 32)

    g1h0, g1h1 = _mm1(features_1, W1, deg1)
    g2h0, g2h1 = _mm1(features_2, W1, deg2)
    s1h0, s1h1, s2h0, s2h1 = _make_layer12()(src1, dst1, src2, dst2,
                                             g1h0, g1h1, g2h0, g2h1)

    q1h0, q1h1 = _mm_mid(s1h0, s1h1, g1h0, g1h1, deg1, b1r, W2, True)
    q2h0, q2h1 = _mm_mid(s2h0, s2h1, g2h0, g2h1, deg2, b1r, W2, True)
    t1h0, t1h1, t2h0, t2h1 = _make_layer12()(src1, dst1, src2, dst2,
                                             q1h0, q1h1, q2h0, q2h1)

    (g31,) = _mm_mid(t1h0, t1h1, q1h0, q1h1, deg1, b2r, W3, False)
    (g32,) = _mm_mid(t2h0, t2h1, q2h0, q2h1, deg2, b2r, W3, False)
    pa0, pa1, pb0, pb1 = _make_layer3()(src1, dst1, src2, dst2, g31, g32)

    a1, a2, m1, m2 = _t1(pa0, pa1, g31, deg1, pb0, pb1, g32, deg2,
                         b3r, Wm1, Wm2)
    ctx1, ctx2 = _t2(a1, a2, m1, m2, Wa)
    p1, p2 = _t3(a1, a2, m1, m2, ctx1, ctx2)

    Wtt = jnp.transpose(Wt, (2, 0, 1))
    out = _t4(p1, p2, Wtt, Wb.T, bt.reshape(1, 16), Wfc, bfc.reshape(1, 16),
              Ws, bs.reshape(1, 1))
    return out


# speedup vs baseline: 16.3646x; 1.1481x over previous
import functools

import jax
import jax.numpy as jnp
from jax import lax
from jax.experimental import pallas as pl
from jax.experimental.pallas import tpu as pltpu
from jax.experimental.pallas import tpu_sc as plsc

NN = 50000
EE = 800000
NPAD = 51200
EPAD = 819200
ECH = EPAD // 128
RPT = NPAD // 16
SCH = 8
BLK = 2000
NBLK = NN // BLK
F32 = jnp.float32

@functools.cache
def _mesh():
    return plsc.VectorSubcoreMesh(core_axis_name="c", subcore_axis_name="s",
                                  num_cores=2, num_subcores=16)


def _zero_fill(buf, width):
    z16 = jnp.zeros((16,), F32)

    def body(i, _):
        for h in range(width // 16):
            buf[i, pl.ds(16 * h, 16)] = z16
        return 0

    lax.fori_loop(0, buf.shape[0], body, 0)


def _unit_scatter(src2d, dst2d, table, out, acc, sbuf, dbuf, rows,
                  gsem, ssem, tile, chunk0, ntile_chunks, sch):
    r0 = tile * RPT
    nb = len(rows)
    gdepth = nb - 2

    _zero_fill(rows[0], rows[0].shape[1])

    zd = [pltpu.async_copy(rows[0], acc.at[pl.ds(r0 + i * 128, 128)], gsem)
          for i in range(RPT // 128)]
    for d in zd:
        d.wait()
    plsc.subcore_barrier()

    nsuper = ntile_chunks // sch

    def sbody(si, _):
        cb = chunk0 + si * sch
        pltpu.sync_copy(src2d.at[pl.ds(cb, sch)], sbuf)
        pltpu.sync_copy(dst2d.at[pl.ds(cb, sch)], dbuf)
        gd = [pltpu.async_copy(table.at[sbuf.at[k]], rows[k], gsem)
              for k in range(gdepth)]
        sd = []
        for j in range(sch):
            gd.pop(0).wait()
            sd.append(pltpu.async_copy(rows[j % nb], acc.at[dbuf.at[j]],
                                       ssem, add=True))
            nj = j + gdepth
            if nj < sch:
                if j >= 2:
                    sd.pop(0).wait()
                gd.append(pltpu.async_copy(table.at[sbuf.at[nj]],
                                           rows[nj % nb], gsem))
        for s in sd:
            s.wait()
        return 0

    lax.fori_loop(0, nsuper, sbody, 0)
    plsc.subcore_barrier()

    def wb(i, _):
        b = rows[0]
        pltpu.sync_copy(acc.at[pl.ds(r0 + i * 256, 128)], b)
        d = pltpu.async_copy(b, out.at[pl.ds(r0 + i * 256, 128)], gsem)
        b2 = rows[1]
        pltpu.sync_copy(acc.at[pl.ds(r0 + i * 256 + 128, 128)], b2)
        d2 = pltpu.async_copy(b2, out.at[pl.ds(r0 + i * 256 + 128, 128)], gsem)
        d.wait()
        d2.wait()
        return 0

    lax.fori_loop(0, RPT // 256, wb, 0)


def _unit_deg(dst2d, out, acc, dbuf, obuf, tile, chunk0, ntile_chunks):
    r0 = tile * RPT

    def zb(i, _):
        pltpu.sync_copy(obuf, acc.at[pl.ds(r0 + i * 128, 128)])
        return 0

    lax.fori_loop(0, RPT // 128, zb, 0)
    plsc.subcore_barrier()

    one16 = jnp.ones((16,), F32)

    def ob(i, _):
        obuf[i, pl.ds(0, 16)] = one16
        return 0

    lax.fori_loop(0, 128, ob, 0)

    nsuper = ntile_chunks // SCH

    def sbody(si, _):
        cb = chunk0 + si * SCH
        pltpu.sync_copy(dst2d.at[pl.ds(cb, SCH)], dbuf)
        for j in range(SCH):
            pltpu.sync_copy(obuf, acc.at[dbuf.at[j]], add=True)
        return 0

    lax.fori_loop(0, nsuper, sbody, 0)
    plsc.subcore_barrier()

    def wb(i, _):
        pltpu.sync_copy(acc.at[pl.ds(r0 + i * 128, 128)], obuf)
        pltpu.sync_copy(obuf, out.at[pl.ds(r0 + i * 128, 128)])
        return 0

    lax.fori_loop(0, RPT // 128, wb, 0)


@functools.cache
def _make_deg():
    @functools.partial(
        pl.kernel,
        out_type=[jax.ShapeDtypeStruct((NPAD, 16), F32)] * 2,
        mesh=_mesh(),
        compiler_params=pltpu.CompilerParams(use_tc_tiling_on_sc=False),
        scratch_types=[
            pltpu.VMEM_SHARED((NPAD + 16, 16), F32),
            pltpu.VMEM((SCH, 128), jnp.int32),
            pltpu.VMEM((128, 16), F32),
        ],
    )
    def _deg_kernel(dst1, dst2, deg1, deg2, acc, dbuf, obuf):
        c = lax.axis_index("c")
        s = lax.axis_index("s")
        _zero_fill(obuf, 16)

        @pl.when(c == 0)
        def _():
            _unit_deg(dst1, deg1, acc, dbuf, obuf, s, s * 400, 400)

        @pl.when(c == 1)
        def _():
            _unit_deg(dst2, deg2, acc, dbuf, obuf, s, s * 400, 400)

    return _deg_kernel


@functools.cache
def _make_layer12():
    @functools.partial(
        pl.kernel,
        out_type=[jax.ShapeDtypeStruct((NPAD, 32), F32)] * 2,
        mesh=_mesh(),
        compiler_params=pltpu.CompilerParams(use_tc_tiling_on_sc=False),
        scratch_types=[
            pltpu.VMEM_SHARED((NPAD + 16, 32), F32),
            pltpu.VMEM((16, 128), jnp.int32),
            pltpu.VMEM((16, 128), jnp.int32),
            pltpu.VMEM((128, 32), F32),
            pltpu.VMEM((128, 32), F32),
            pltpu.VMEM((128, 32), F32),
            pltpu.VMEM((128, 32), F32),
            pltpu.VMEM((128, 32), F32),
            pltpu.SemaphoreType.DMA,
            pltpu.SemaphoreType.DMA,
        ],
    )
    def _k(src1, dst1, gh0, gh1, oh0, oh1,
           acc, sbuf, dbuf, rows0, rows1, rows2, rows3, rows4, gsem, ssem):
        c = lax.axis_index("c")
        s = lax.axis_index("s")
        rows = (rows0, rows1, rows2, rows3, rows4)

        @pl.when(c == 0)
        def _():
            _unit_scatter(src1, dst1, gh0, oh0, acc, sbuf, dbuf, rows,
                          gsem, ssem, s, s * 400, 400, 16)

        @pl.when(c == 1)
        def _():
            _unit_scatter(src1, dst1, gh1, oh1, acc, sbuf, dbuf, rows,
                          gsem, ssem, s, s * 400, 400, 16)

    return _k


@functools.cache
def _make_layer3():
    @functools.partial(
        pl.kernel,
        out_type=[jax.ShapeDtypeStruct((NPAD, 32), F32)] * 2,
        mesh=_mesh(),
        compiler_params=pltpu.CompilerParams(use_tc_tiling_on_sc=False),
        scratch_types=[
            pltpu.VMEM_SHARED((NPAD + 16, 32), F32),
            pltpu.VMEM((8, 128), jnp.int32),
            pltpu.VMEM((8, 128), jnp.int32),
            pltpu.VMEM((128, 32), F32),
            pltpu.VMEM((128, 32), F32),
            pltpu.VMEM((128, 32), F32),
            pltpu.VMEM((128, 32), F32),
            pltpu.VMEM((128, 32), F32),
            pltpu.SemaphoreType.DMA,
            pltpu.SemaphoreType.DMA,
        ],
    )
    def _layer3_kernel(src1, dst1, g3a, pa0, pa1,
                       acc, sbuf, dbuf, rows0, rows1, rows2, rows3, rows4,
                       gsem, ssem):
        c = lax.axis_index("c")
        s = lax.axis_index("s")
        rows = (rows0, rows1, rows2, rows3, rows4)

        @pl.when(c == 0)
        def _():
            _unit_scatter(src1, dst1, g3a, pa0, acc, sbuf, dbuf, rows,
                          gsem, ssem, s, s * 200, 200, 8)

        @pl.when(c == 1)
        def _():
            _unit_scatter(src1, dst1, g3a, pa1, acc, sbuf, dbuf, rows,
                          gsem, ssem, s, 3200 + s * 200, 200, 8)

    return _layer3_kernel


def _mm1_body(x_ref, w_ref, deg_ref, o0_ref, o1_ref):
    dinv = lax.rsqrt(deg_ref[:, 0:1] + 1.0)
    h = jnp.dot(x_ref[...], w_ref[...], preferred_element_type=F32) * dinv
    o0_ref[...] = h[:, :32]
    o1_ref[...] = h[:, 32:]


def _mm1(x, W, deg):
    return pl.pallas_call(
        _mm1_body,
        grid=(NBLK,),
        in_specs=[
            pl.BlockSpec((BLK, 128), lambda i: (i, 0)),
            pl.BlockSpec((128, 64), lambda i: (0, 0)),
            pl.BlockSpec((BLK, 16), lambda i: (i, 0)),
        ],
        out_specs=[pl.BlockSpec((BLK, 32), lambda i: (i, 0))] * 2,
        out_shape=[jax.ShapeDtypeStruct((NN, 32), F32)] * 2,
    )(x, W, deg)


def _mm_mid_body2(s0, s1, g0, g1, deg, b, w, o0, o1):
    dinv = lax.rsqrt(deg[:, 0:1] + 1.0)
    prev = jnp.concatenate([s0[...] + g0[...], s1[...] + g1[...]], axis=1)
    x = jnp.maximum(prev * dinv + b[...], 0.0)
    h = jnp.dot(x, w[...], preferred_element_type=F32) * dinv
    o0[...] = h[:, :32]
    o1[...] = h[:, 32:]


def _mm_mid_body1(s0, s1, g0, g1, deg, b, w, o0):
    dinv = lax.rsqrt(deg[:, 0:1] + 1.0)
    prev = jnp.concatenate([s0[...] + g0[...], s1[...] + g1[...]], axis=1)
    x = jnp.maximum(prev * dinv + b[...], 0.0)
    o0[...] = jnp.dot(x, w[...], preferred_element_type=F32) * dinv


def _mm_mid(s0, s1, g0, g1, deg, b_row, W, halves):
    fout = W.shape[1]
    nouts = 2 if halves else 1
    return pl.pallas_call(
        _mm_mid_body2 if halves else _mm_mid_body1,
        grid=(NBLK,),
        in_specs=[
            pl.BlockSpec((BLK, 32), lambda i: (i, 0)),
            pl.BlockSpec((BLK, 32), lambda i: (i, 0)),
            pl.BlockSpec((BLK, 32), lambda i: (i, 0)),
            pl.BlockSpec((BLK, 32), lambda i: (i, 0)),
            pl.BlockSpec((BLK, 16), lambda i: (i, 0)),
            pl.BlockSpec((1, 64), lambda i: (0, 0)),
            pl.BlockSpec((64, fout), lambda i: (0, 0)),
        ],
        out_specs=[pl.BlockSpec((BLK, fout // nouts), lambda i: (i, 0))] * nouts,
        out_shape=[jax.ShapeDtypeStruct((NN, fout // nouts), F32)] * nouts,
    )(s0, s1, g0, g1, deg, b_row, W)


def _t1_body(p10, p11, g31, deg1, p20, p21, g32, deg2, b3, wm1, wm2,
             a1_ref, a2_ref, m1_ref, m2_ref, acc1, acc2):
    i = pl.program_id(0)

    @pl.when(i == 0)
    def _():
        acc1[...] = jnp.zeros_like(acc1)
        acc2[...] = jnp.zeros_like(acc2)

    dinv1 = lax.rsqrt(deg1[:, 0:1] + 1.0)
    a1 = (p10[...] + p11[...] + g31[...]) * dinv1 + b3[...]
    dinv2 = lax.rsqrt(deg2[:, 0:1] + 1.0)
    a2 = (p20[...] + p21[...] + g32[...]) * dinv2 + b3[...]
    a1_ref[...] = a1
    a2_ref[...] = a2
    acc1[...] += jnp.sum(a1, axis=0, keepdims=True)
    acc2[...] += jnp.sum(a2, axis=0, keepdims=True)
    inv_n = 1.0 / NN
    m1_ref[...] = jnp.tanh(
        jnp.dot(acc1[...] * inv_n, wm1[...], preferred_element_type=F32))
    m2_ref[...] = jnp.tanh(
        jnp.dot(acc2[...] * inv_n, wm2[...], preferred_element_type=F32))


def _t1(p10, p11, g31, deg1, p20, p21, g32, deg2, b3_row, Wm1, Wm2):
    blk = pl.BlockSpec((BLK, 32), lambda i: (i, 0))
    small = pl.BlockSpec((1, 32), lambda i: (0, 0))
    return pl.pallas_call(
        _t1_body,
        grid=(NBLK,),
        in_specs=[blk, blk, blk, pl.BlockSpec((BLK, 16), lambda i: (i, 0)),
                  blk, blk, blk, pl.BlockSpec((BLK, 16), lambda i: (i, 0)),
                  small,
                  pl.BlockSpec((32, 32), lambda i: (0, 0)),
                  pl.BlockSpec((32, 32), lambda i: (0, 0))],
        out_specs=[blk, blk, small, small],
        out_shape=[jax.ShapeDtypeStruct((NN, 32), F32),
                   jax.ShapeDtypeStruct((NN, 32), F32),
                   jax.ShapeDtypeStruct((1, 32), F32),
                   jax.ShapeDtypeStruct((1, 32), F32)],
        scratch_shapes=[pltpu.VMEM((1, 32), F32), pltpu.VMEM((1, 32), F32)],
    )(p10, p11, g31, deg1, p20, p21, g32, deg2, b3_row, Wm1, Wm2)


def _t2_body(a1, a2, m1, m2, wa, ctx1_ref, ctx2_ref, acc1, acc2):
    i = pl.program_id(0)

    @pl.when(i == 0)
    def _():
        acc1[...] = jnp.zeros_like(acc1)
        acc2[...] = jnp.zeros_like(acc2)

    x1 = jnp.abs(a1[...] - m2[...])
    x2 = jnp.abs(a2[...] - m1[...])
    acc1[...] += jnp.sum(x1, axis=0, keepdims=True)
    acc2[...] += jnp.sum(x2, axis=0, keepdims=True)
    inv_n = 1.0 / NN
    ctx1_ref[...] = jnp.tanh(
        jnp.dot(acc1[...] * inv_n, wa[...], preferred_element_type=F32))
    ctx2_ref[...] = jnp.tanh(
        jnp.dot(acc2[...] * inv_n, wa[...], preferred_element_type=F32))


def _t2(a1, a2, m1, m2, Wa):
    blk = pl.BlockSpec((BLK, 32), lambda i: (i, 0))
    small = pl.BlockSpec((1, 32), lambda i: (0, 0))
    return pl.pallas_call(
        _t2_body,
        grid=(NBLK,),
        in_specs=[blk, blk, small, small,
                  pl.BlockSpec((32, 32), lambda i: (0, 0))],
        out_specs=[small, small],
        out_shape=[jax.ShapeDtypeStruct((1, 32), F32)] * 2,
        scratch_shapes=[pltpu.VMEM((1, 32), F32), pltpu.VMEM((1, 32), F32)],
    )(a1, a2, m1, m2, Wa)


def _t3_body(a1, a2, m1, m2, ctx1, ctx2, p1_ref, p2_ref, acc1, acc2):
    i = pl.program_id(0)

    @pl.when(i == 0)
    def _():
        acc1[...] = jnp.zeros_like(acc1)
        acc2[...] = jnp.zeros_like(acc2)

    x1 = jnp.abs(a1[...] - m2[...])
    x2 = jnp.abs(a2[...] - m1[...])
    s1 = jax.nn.sigmoid(jnp.sum(x1 * ctx1[...], axis=1, keepdims=True))
    s2 = jax.nn.sigmoid(jnp.sum(x2 * ctx2[...], axis=1, keepdims=True))
    acc1[...] += jnp.sum(x1 * s1, axis=0, keepdims=True)
    acc2[...] += jnp.sum(x2 * s2, axis=0, keepdims=True)
    p1_ref[...] = acc1[...]
    p2_ref[...] = acc2[...]


def _t3(a1, a2, m1, m2, ctx1, ctx2):
    blk = pl.BlockSpec((BLK, 32), lambda i: (i, 0))
    small = pl.BlockSpec((1, 32), lambda i: (0, 0))
    return pl.pallas_call(
        _t3_body,
        grid=(NBLK,),
        in_specs=[blk, blk, small, small, small, small],
        out_specs=[small, small],
        out_shape=[jax.ShapeDtypeStruct((1, 32), F32)] * 2,
        scratch_shapes=[pltpu.VMEM((1, 32), F32), pltpu.VMEM((1, 32), F32)],
    )(a1, a2, m1, m2, ctx1, ctx2)


def _t4_body(p1, p2, wtt, wbt, btr, wfc, bfcr, ws, bsr, out_ref, sc):
    t = pl.program_id(0)
    a = wtt[0]
    v = jnp.dot(p1[...], a, preferred_element_type=F32)
    s_t = jnp.sum(v * p2[...])
    iota = lax.broadcasted_iota(jnp.int32, (1, 16), 1)
    base = jnp.where(t == 0, jnp.zeros_like(sc[...]), sc[...])
    sc[...] = jnp.where(iota == t, s_t, base)
    combined = jnp.concatenate([p1[...], p2[...]], axis=1)
    block = jnp.dot(combined, wbt[...], preferred_element_type=F32)
    scores = jnp.maximum(sc[...] + block + btr[...], 0.0)
    h = jnp.maximum(
        jnp.dot(scores, wfc[...], preferred_element_type=F32) + bfcr[...], 0.0)
    out_ref[...] = jax.nn.sigmoid(
        jnp.dot(h, ws[...], preferred_element_type=F32) + bsr[...])


def _t4(p1, p2, Wtt, Wbt, btr, Wfc, bfcr, Ws, bsr):
    small = pl.BlockSpec((1, 32), lambda t: (0, 0))
    return pl.pallas_call(
        _t4_body,
        grid=(16,),
        in_specs=[small, small,
                  pl.BlockSpec((1, 32, 32), lambda t: (t, 0, 0)),
                  pl.BlockSpec((64, 16), lambda t: (0, 0)),
                  pl.BlockSpec((1, 16), lambda t: (0, 0)),
                  pl.BlockSpec((16, 16), lambda t: (0, 0)),
                  pl.BlockSpec((1, 16), lambda t: (0, 0)),
                  pl.BlockSpec((16, 1), lambda t: (0, 0)),
                  pl.BlockSpec((1, 1), lambda t: (0, 0))],
        out_specs=pl.BlockSpec((1, 1), lambda t: (0, 0)),
        out_shape=jax.ShapeDtypeStruct((1, 1), F32),
        scratch_shapes=[pltpu.VMEM((1, 16), F32)],
    )(p1, p2, Wtt, Wbt, btr, Wfc, bfcr, Ws, bsr)


def _pad_edges(ei):
    pad = EPAD - EE
    src = jnp.concatenate([ei[0], jnp.zeros((pad,), jnp.int32)])
    dst = jnp.concatenate([ei[1], jnp.full((pad,), NPAD, jnp.int32)])
    return src.reshape(ECH, 128), dst.reshape(ECH, 128)


def kernel(features_1, edge_index_1, features_2, edge_index_2, W1, b1, W2, b2,
           W3, b3, Wm1, Wm2, Wa, Wt, Wb, bt, Wfc, bfc, Ws, bs):
    src1, dst1 = _pad_edges(edge_index_1)
    src2, dst2 = _pad_edges(edge_index_2)

    deg1, deg2 = _make_deg()(dst1, dst2)

    b1r = b1.reshape(1, 64)
    b2r = b2.reshape(1, 64)
    b3r = b3.reshape(1, 32)

    g1h0, g1h1 = _mm1(features_1, W1, deg1)
    g2h0, g2h1 = _mm1(features_2, W1, deg2)
    s1h0, s1h1 = _make_layer12()(src1, dst1, g1h0, g1h1)
    s2h0, s2h1 = _make_layer12()(src2, dst2, g2h0, g2h1)

    q1h0, q1h1 = _mm_mid(s1h0, s1h1, g1h0, g1h1, deg1, b1r, W2, True)
    q2h0, q2h1 = _mm_mid(s2h0, s2h1, g2h0, g2h1, deg2, b1r, W2, True)
    t1h0, t1h1 = _make_layer12()(src1, dst1, q1h0, q1h1)
    t2h0, t2h1 = _make_layer12()(src2, dst2, q2h0, q2h1)

    (g31,) = _mm_mid(t1h0, t1h1, q1h0, q1h1, deg1, b2r, W3, False)
    (g32,) = _mm_mid(t2h0, t2h1, q2h0, q2h1, deg2, b2r, W3, False)
    pa0, pa1 = _make_layer3()(src1, dst1, g31)
    pb0, pb1 = _make_layer3()(src2, dst2, g32)

    a1, a2, m1, m2 = _t1(pa0, pa1, g31, deg1, pb0, pb1, g32, deg2,
                         b3r, Wm1, Wm2)
    ctx1, ctx2 = _t2(a1, a2, m1, m2, Wa)
    p1, p2 = _t3(a1, a2, m1, m2, ctx1, ctx2)

    Wtt = jnp.transpose(Wt, (2, 0, 1))
    out = _t4(p1, p2, Wtt, Wb.T, bt.reshape(1, 16), Wfc, bfc.reshape(1, 16),
              Ws, bs.reshape(1, 1))
    return out
